# baseline (speedup 1.0000x reference)
import functools

import jax
import jax.numpy as jnp
from jax import lax
from jax.experimental import pallas as pl
from jax.experimental.pallas import tpu as pltpu

_INTERPRET = False

N_E = 800000
N_V = 50000
NG = 64
BR_E = 2000
BR_N = 2000
f32 = jnp.float32


def _row(v):
    return v.reshape(1, -1)


def _fin_stats(s, ss, n, g, bbn):
    mu = s / n
    var = ss / n - mu * mu
    isd = 1.0 / jnp.sqrt(var + 1e-5)
    sc = g * isd
    sh = bbn - mu * sc
    return _row(sc), _row(sh)


def _dot(a, b):
    return jnp.dot(a, b, preferred_element_type=f32)


def _full(shape):
    return pl.BlockSpec(shape, lambda *a: tuple(0 for _ in shape))


def _rows(br, w):
    return pl.BlockSpec((br, w), lambda i: (i, 0))


def _sd(shape):
    return jax.ShapeDtypeStruct(shape, f32)


def _p0_body(src, dst, ea, w1s, w1d, w1e, b1, s_ref, ss_ref):
    h = (_dot(src[...], w1s[...]) + _dot(dst[...], w1d[...])
         + _dot(ea[...], w1e[...]) + b1[...])

    @pl.when(pl.program_id(0) == 0)
    def _():
        s_ref[...] = jnp.zeros_like(s_ref)
        ss_ref[...] = jnp.zeros_like(ss_ref)

    s_ref[...] += jnp.sum(h, axis=0, keepdims=True)
    ss_ref[...] += jnp.sum(h * h, axis=0, keepdims=True)


def _edge_stats1(src, dst, ea, w1s, w1d, w1e, b1):
    ls = w1s.shape[1]
    s, ss = pl.pallas_call(
        _p0_body,
        grid=(N_E // BR_E,),
        in_specs=[_rows(BR_E, 16), _rows(BR_E, 16), _rows(BR_E, 19),
                  _full((16, ls)), _full((16, ls)), _full((19, ls)),
                  _full((1, ls))],
        out_specs=[_full((1, ls)), _full((1, ls))],
        out_shape=[_sd((1, ls))] * 2,
        interpret=_INTERPRET,
    )(src, dst, ea, w1s, w1d, w1e, _row(b1))
    return s, ss


def _p1_body(src, dst, ea, w1s, w1d, w1e, b1, sc1, sh1, w2, b2,
             h2_ref, s_ref, ss_ref):
    h1 = (_dot(src[...], w1s[...]) + _dot(dst[...], w1d[...])
          + _dot(ea[...], w1e[...]) + b1[...])
    a1 = jnp.maximum(h1 * sc1[...] + sh1[...], 0.0)
    h2 = _dot(a1, w2[...]) + b2[...]
    h2_ref[...] = h2

    @pl.when(pl.program_id(0) == 0)
    def _():
        s_ref[...] = jnp.zeros_like(s_ref)
        ss_ref[...] = jnp.zeros_like(ss_ref)

    s_ref[...] += jnp.sum(h2, axis=0, keepdims=True)
    ss_ref[...] += jnp.sum(h2 * h2, axis=0, keepdims=True)


def _edge_p1(src, dst, ea, w1s, w1d, w1e, b1, sc1, sh1, w2, b2):
    ls = w2.shape[1]
    return pl.pallas_call(
        _p1_body,
        grid=(N_E // BR_E,),
        in_specs=[_rows(BR_E, 16), _rows(BR_E, 16), _rows(BR_E, 19),
                  _full((16, ls)), _full((16, ls)), _full((19, ls)),
                  _full((1, ls)), _full((1, ls)), _full((1, ls)),
                  _full((ls, ls)), _full((1, ls))],
        out_specs=[_rows(BR_E, ls), _full((1, ls)), _full((1, ls))],
        out_shape=[_sd((N_E, ls)), _sd((1, ls)), _sd((1, ls))],
        interpret=_INTERPRET,
    )(src, dst, ea, w1s, w1d, w1e, _row(b1), sc1, sh1, w2, _row(b2))


def _p2_body(h2, src, sc2, sh2, w3, b3, wm1x, wm1e, bm1,
             ea_ref, m1_ref, s_ref, ss_ref):
    a2 = jnp.maximum(h2[...] * sc2[...] + sh2[...], 0.0)
    ea = jnp.maximum(_dot(a2, w3[...]) + b3[...], 0.0)
    ea_ref[...] = ea
    m1 = _dot(src[...], wm1x[...]) + _dot(ea, wm1e[...]) + bm1[...]
    m1_ref[...] = m1

    @pl.when(pl.program_id(0) == 0)
    def _():
        s_ref[...] = jnp.zeros_like(s_ref)
        ss_ref[...] = jnp.zeros_like(ss_ref)

    s_ref[...] += jnp.sum(m1, axis=0, keepdims=True)
    ss_ref[...] += jnp.sum(m1 * m1, axis=0, keepdims=True)


def _p2_body_head(h2, src, sc2, sh2, w3, b3, wm1x, wm1e, bm1, we, be,
                  ea_ref, m1_ref, s_ref, ss_ref, hd_ref):
    a2 = jnp.maximum(h2[...] * sc2[...] + sh2[...], 0.0)
    ea = jnp.maximum(_dot(a2, w3[...]) + b3[...], 0.0)
    ea_ref[...] = ea
    m1 = _dot(src[...], wm1x[...]) + _dot(ea, wm1e[...]) + bm1[...]
    m1_ref[...] = m1

    @pl.when(pl.program_id(0) == 0)
    def _():
        s_ref[...] = jnp.zeros_like(s_ref)
        ss_ref[...] = jnp.zeros_like(ss_ref)

    s_ref[...] += jnp.sum(m1, axis=0, keepdims=True)
    ss_ref[...] += jnp.sum(m1 * m1, axis=0, keepdims=True)
    hd_ref[...] = jax.nn.sigmoid(_dot(ea, we[...]) + be[...])


def _edge_p2(h2, src, sc2, sh2, w3, b3, wm1x, wm1e, bm1, we=None, be=None):
    ls = h2.shape[1]
    lsn = wm1x.shape[1]
    ins = [_rows(BR_E, ls), _rows(BR_E, 16), _full((1, ls)), _full((1, ls)),
           _full((ls, 19)), _full((1, 19)), _full((16, lsn)),
           _full((19, lsn)), _full((1, lsn))]
    outs = [_rows(BR_E, 19), _rows(BR_E, lsn), _full((1, lsn)),
            _full((1, lsn))]
    oshapes = [_sd((N_E, 19)), _sd((N_E, lsn)), _sd((1, lsn)), _sd((1, lsn))]
    args = [h2, src, sc2, sh2, w3, _row(b3), wm1x, wm1e, _row(bm1)]
    if we is None:
        body = _p2_body
    else:
        body = _p2_body_head
        ins += [_full((19, 1)), _full((1, 1))]
        outs.append(_rows(BR_E, 1))
        oshapes.append(_sd((N_E, 1)))
        args += [we, _row(be)]
    return pl.pallas_call(
        body, grid=(N_E // BR_E,), in_specs=ins, out_specs=outs,
        out_shape=oshapes, interpret=_INTERPRET,
    )(*args)


def _p4_body(x, seg, cnt, wm2x, wm2s, bm2, m2_ref, s_ref, ss_ref):
    c = jnp.maximum(cnt[...][:, 0:1], 1.0)
    segm = seg[...] * (1.0 / c)
    m2 = _dot(x[...], wm2x[...]) + _dot(segm, wm2s[...]) + bm2[...]
    m2_ref[...] = m2

    @pl.when(pl.program_id(0) == 0)
    def _():
        s_ref[...] = jnp.zeros_like(s_ref)
        ss_ref[...] = jnp.zeros_like(ss_ref)

    s_ref[...] += jnp.sum(m2, axis=0, keepdims=True)
    ss_ref[...] += jnp.sum(m2 * m2, axis=0, keepdims=True)


def _node_p4(x, seg, cnt16, wm2x, wm2s, bm2):
    ls = seg.shape[1]
    e2 = wm2x.shape[1]
    return pl.pallas_call(
        _p4_body,
        grid=(N_V // BR_N,),
        in_specs=[_rows(BR_N, 16), _rows(BR_N, ls), _rows(BR_N, 16),
                  _full((16, e2)), _full((ls, e2)), _full((1, e2))],
        out_specs=[_rows(BR_N, e2), _full((1, e2)), _full((1, e2))],
        out_shape=[_sd((N_V, e2)), _sd((1, e2)), _sd((1, e2))],
        interpret=_INTERPRET,
    )(x, seg, cnt16, wm2x, wm2s, _row(bm2))


def _p5_body_bsum(m2, oh, sc, sh, wlin, blin, x_ref, bs_ref, bc_ref):
    a = jnp.maximum(m2[...] * sc[...] + sh[...], 0.0)
    xn = jnp.maximum(_dot(a, wlin[...]) + blin[...], 0.0)
    x_ref[...] = xn

    @pl.when(pl.program_id(0) == 0)
    def _():
        bs_ref[...] = jnp.zeros_like(bs_ref)
        bc_ref[...] = jnp.zeros_like(bc_ref)

    o = oh[...]
    bs_ref[...] += lax.dot_general(o, xn, (((0,), (0,)), ((), ())),
                                   preferred_element_type=f32)
    ones = jnp.ones((o.shape[0], 1), f32)
    bc_ref[...] += lax.dot_general(o, ones, (((0,), (0,)), ((), ())),
                                   preferred_element_type=f32)


def _p5_body_head(m2, sc, sh, wlin, blin, wx, bx, x_ref, y_ref):
    a = jnp.maximum(m2[...] * sc[...] + sh[...], 0.0)
    xn = jnp.maximum(_dot(a, wlin[...]) + blin[...], 0.0)
    x_ref[...] = xn
    y_ref[...] = jax.nn.sigmoid(_dot(xn, wx[...]) + bx[...])


def _node_p5(m2, sc, sh, wlin, blin, onehot=None, wx=None, bx=None):
    e2 = m2.shape[1]
    if onehot is not None:
        return pl.pallas_call(
            _p5_body_bsum,
            grid=(N_V // BR_N,),
            in_specs=[_rows(BR_N, e2), _rows(BR_N, NG), _full((1, e2)),
                      _full((1, e2)), _full((e2, 16)), _full((1, 16))],
            out_specs=[_rows(BR_N, 16), _full((NG, 16)), _full((NG, 1))],
            out_shape=[_sd((N_V, 16)), _sd((NG, 16)), _sd((NG, 1))],
            interpret=_INTERPRET,
        )(m2, onehot, sc, sh, wlin, _row(blin))
    return pl.pallas_call(
        _p5_body_head,
        grid=(N_V // BR_N,),
        in_specs=[_rows(BR_N, e2), _full((1, e2)), _full((1, e2)),
                  _full((e2, 16)), _full((1, 16)), _full((16, 1)),
                  _full((1, 1))],
        out_specs=[_rows(BR_N, 16), _rows(BR_N, 1)],
        out_shape=[_sd((N_V, 16)), _sd((N_V, 1))],
        interpret=_INTERPRET,
    )(m2, sc, sh, wlin, _row(blin), wx, _row(bx))


def _p6_body(u, bs, bc, w1u, w1s, b1, g1, bb1, w2, b2, g2, bb2, w3, b3,
             out_ref):
    cnt = jnp.maximum(bc[...], 1.0)
    seg = bs[...] * (1.0 / cnt)
    h = _dot(u[...], w1u[...]) + _dot(seg, w1s[...]) + b1[...]

    def _bn(h, g, bb):
        mu = jnp.mean(h, axis=0, keepdims=True)
        var = jnp.mean(h * h, axis=0, keepdims=True) - mu * mu
        return (h - mu) / jnp.sqrt(var + 1e-5) * g[...] + bb[...]

    a1 = jnp.maximum(_bn(h, g1, bb1), 0.0)
    h2 = _dot(a1, w2[...]) + b2[...]
    a2 = jnp.maximum(_bn(h2, g2, bb2), 0.0)
    out_ref[...] = _dot(a2, w3[...]) + b3[...]


def _glob_p6(u, bs, bc, gp):
    w1u, w1s = gp["l1"]["w"][:7], gp["l1"]["w"][7:]
    return pl.pallas_call(
        _p6_body,
        in_specs=[_full((NG, 7)), _full((NG, 16)), _full((NG, 1)),
                  _full((7, 128)), _full((16, 128)), _full((1, 128)),
                  _full((1, 128)), _full((1, 128)), _full((128, 128)),
                  _full((1, 128)), _full((1, 128)), _full((1, 128)),
                  _full((128, 7)), _full((1, 7))],
        out_specs=_full((NG, 7)),
        out_shape=_sd((NG, 7)),
        interpret=_INTERPRET,
    )(u, bs, bc, w1u, w1s, _row(gp["l1"]["b"]), _row(gp["bn1"]["g"]),
      _row(gp["bn1"]["b"]), gp["l2"]["w"], _row(gp["l2"]["b"]),
      _row(gp["bn2"]["g"]), _row(gp["bn2"]["b"]), gp["l3"]["w"],
      _row(gp["l3"]["b"]))


def _gather(x, idx):
    return x[idx]


def _seg_relu_bn_sum(m1, col, scn, shn):
    h = jnp.maximum(m1 * scn + shn, 0.0)
    return jax.ops.segment_sum(h, col, num_segments=N_V)


def _seg_counts(col):
    c = jax.ops.segment_sum(jnp.ones((N_E,), f32), col, num_segments=N_V)
    return jnp.broadcast_to(c[:, None], (N_V, 16))


def kernel(x, edge_index, edge_attr, u, batch, params):
    row, col = edge_index[0], edge_index[1]
    onehot = (batch[:, None] == jnp.arange(NG, dtype=jnp.int32)[None, :]
              ).astype(f32)
    cnt16 = _seg_counts(col)
    ea = edge_attr
    y_pred = None
    head = None
    for li, name in enumerate(("ml1", "ml2", "ml3")):
        p = params[name]
        ep, np_, gp = p["edge"], p["node"], p["glob"]
        last = li == 2
        src = _gather(x, row)
        dst = _gather(x, col)
        w1 = ep["l1"]["w"]
        w1s, w1d, w1e = w1[:16], w1[16:32], w1[32:]
        s1, ss1 = _edge_stats1(src, dst, ea, w1s, w1d, w1e, ep["l1"]["b"])
        sc1, sh1 = _fin_stats(s1[0], ss1[0], N_E, ep["bn1"]["g"],
                              ep["bn1"]["b"])
        h2, s2, ss2 = _edge_p1(src, dst, ea, w1s, w1d, w1e, ep["l1"]["b"],
                               sc1, sh1, ep["l2"]["w"], ep["l2"]["b"])
        sc2, sh2 = _fin_stats(s2[0], ss2[0], N_E, ep["bn2"]["g"],
                              ep["bn2"]["b"])
        wm1 = np_["m1"]["w"]
        wm1x, wm1e = wm1[:16], wm1[16:]
        if last:
            ea, m1, sn, ssn, head = _edge_p2(
                h2, src, sc2, sh2, ep["lin"]["w"], ep["lin"]["b"],
                wm1x, wm1e, np_["m1"]["b"],
                params["e_lin"]["w"], params["e_lin"]["b"])
        else:
            ea, m1, sn, ssn = _edge_p2(
                h2, src, sc2, sh2, ep["lin"]["w"], ep["lin"]["b"],
                wm1x, wm1e, np_["m1"]["b"])
        scn, shn = _fin_stats(sn[0], ssn[0], N_E, np_["bn1"]["g"],
                              np_["bn1"]["b"])
        seg = _seg_relu_bn_sum(m1, col, scn, shn)
        wm2 = np_["m2"]["w"]
        m2, sm, ssm = _node_p4(x, seg, cnt16, wm2[:16], wm2[16:],
                               np_["m2"]["b"])
        scm, shm = _fin_stats(sm[0], ssm[0], N_V, np_["bn2"]["g"],
                              np_["bn2"]["b"])
        if last:
            x, y_pred = _node_p5(m2, scm, shm, np_["lin"]["w"],
                                 np_["lin"]["b"], wx=params["x_lin"]["w"],
                                 bx=params["x_lin"]["b"])
        else:
            x, bs, bc = _node_p5(m2, scm, shm, np_["lin"]["w"],
                                 np_["lin"]["b"], onehot=onehot)
            u = _glob_p6(u, bs, bc, gp)
    return (y_pred, head)


# speedup vs baseline: 1.8695x; 1.8695x over previous
import functools

import jax
import jax.numpy as jnp
from jax import lax
from jax.experimental import pallas as pl
from jax.experimental.pallas import tpu as pltpu
from jax.experimental.pallas import tpu_sc as plsc

_INTERPRET = False

_NC = 2
_NS = 16
_NW = _NC * _NS
_G_NCH = 800000 // 128

N_E = 800000
N_V = 50000
NG = 64
BR_E = 2000
BR_N = 2000
f32 = jnp.float32


def _row(v):
    return v.reshape(1, -1)


def _fin_stats(s, ss, n, g, bbn):
    mu = s / n
    var = ss / n - mu * mu
    isd = 1.0 / jnp.sqrt(var + 1e-5)
    sc = g * isd
    sh = bbn - mu * sc
    return _row(sc), _row(sh)


def _dot(a, b):
    return jnp.dot(a, b, preferred_element_type=f32)


def _full(shape):
    return pl.BlockSpec(shape, lambda *a: tuple(0 for _ in shape))


def _rows(br, w):
    return pl.BlockSpec((br, w), lambda i: (i, 0))


def _sd(shape):
    return jax.ShapeDtypeStruct(shape, f32)


def _p0_body(src, dst, ea, w1s, w1d, w1e, b1, s_ref, ss_ref):
    h = (_dot(src[...], w1s[...]) + _dot(dst[...], w1d[...])
         + _dot(ea[...], w1e[...]) + b1[...])

    @pl.when(pl.program_id(0) == 0)
    def _():
        s_ref[...] = jnp.zeros_like(s_ref)
        ss_ref[...] = jnp.zeros_like(ss_ref)

    s_ref[...] += jnp.sum(h, axis=0, keepdims=True)
    ss_ref[...] += jnp.sum(h * h, axis=0, keepdims=True)


def _edge_stats1(src, dst, ea, w1s, w1d, w1e, b1):
    ls = w1s.shape[1]
    s, ss = pl.pallas_call(
        _p0_body,
        grid=(N_E // BR_E,),
        in_specs=[_rows(BR_E, 16), _rows(BR_E, 16), _rows(BR_E, 19),
                  _full((16, ls)), _full((16, ls)), _full((19, ls)),
                  _full((1, ls))],
        out_specs=[_full((1, ls)), _full((1, ls))],
        out_shape=[_sd((1, ls))] * 2,
        interpret=_INTERPRET,
    )(src, dst, ea, w1s, w1d, w1e, _row(b1))
    return s, ss


def _p1_body(src, dst, ea, w1s, w1d, w1e, b1, sc1, sh1, w2, b2,
             h2_ref, s_ref, ss_ref):
    h1 = (_dot(src[...], w1s[...]) + _dot(dst[...], w1d[...])
          + _dot(ea[...], w1e[...]) + b1[...])
    a1 = jnp.maximum(h1 * sc1[...] + sh1[...], 0.0)
    h2 = _dot(a1, w2[...]) + b2[...]
    h2_ref[...] = h2

    @pl.when(pl.program_id(0) == 0)
    def _():
        s_ref[...] = jnp.zeros_like(s_ref)
        ss_ref[...] = jnp.zeros_like(ss_ref)

    s_ref[...] += jnp.sum(h2, axis=0, keepdims=True)
    ss_ref[...] += jnp.sum(h2 * h2, axis=0, keepdims=True)


def _edge_p1(src, dst, ea, w1s, w1d, w1e, b1, sc1, sh1, w2, b2):
    ls = w2.shape[1]
    return pl.pallas_call(
        _p1_body,
        grid=(N_E // BR_E,),
        in_specs=[_rows(BR_E, 16), _rows(BR_E, 16), _rows(BR_E, 19),
                  _full((16, ls)), _full((16, ls)), _full((19, ls)),
                  _full((1, ls)), _full((1, ls)), _full((1, ls)),
                  _full((ls, ls)), _full((1, ls))],
        out_specs=[_rows(BR_E, ls), _full((1, ls)), _full((1, ls))],
        out_shape=[_sd((N_E, ls)), _sd((1, ls)), _sd((1, ls))],
        interpret=_INTERPRET,
    )(src, dst, ea, w1s, w1d, w1e, _row(b1), sc1, sh1, w2, _row(b2))


def _p2_body(h2, src, sc2, sh2, w3, b3, wm1x, wm1e, bm1,
             ea_ref, s_ref, ss_ref):
    a2 = jnp.maximum(h2[...] * sc2[...] + sh2[...], 0.0)
    ea = jnp.maximum(_dot(a2, w3[...]) + b3[...], 0.0)
    ea_ref[...] = ea
    m1 = _dot(src[...], wm1x[...]) + _dot(ea, wm1e[...]) + bm1[...]

    @pl.when(pl.program_id(0) == 0)
    def _():
        s_ref[...] = jnp.zeros_like(s_ref)
        ss_ref[...] = jnp.zeros_like(ss_ref)

    s_ref[...] += jnp.sum(m1, axis=0, keepdims=True)
    ss_ref[...] += jnp.sum(m1 * m1, axis=0, keepdims=True)


def _p2_body_head(h2, src, sc2, sh2, w3, b3, wm1x, wm1e, bm1, we, be,
                  ea_ref, s_ref, ss_ref, hd_ref):
    a2 = jnp.maximum(h2[...] * sc2[...] + sh2[...], 0.0)
    ea = jnp.maximum(_dot(a2, w3[...]) + b3[...], 0.0)
    ea_ref[...] = ea
    m1 = _dot(src[...], wm1x[...]) + _dot(ea, wm1e[...]) + bm1[...]

    @pl.when(pl.program_id(0) == 0)
    def _():
        s_ref[...] = jnp.zeros_like(s_ref)
        ss_ref[...] = jnp.zeros_like(ss_ref)

    s_ref[...] += jnp.sum(m1, axis=0, keepdims=True)
    ss_ref[...] += jnp.sum(m1 * m1, axis=0, keepdims=True)
    hd_ref[...] = jax.nn.sigmoid(_dot(ea, we[...]) + be[...])


def _edge_p2(h2, src, sc2, sh2, w3, b3, wm1x, wm1e, bm1, we=None, be=None):
    ls = h2.shape[1]
    lsn = wm1x.shape[1]
    ins = [_rows(BR_E, ls), _rows(BR_E, 16), _full((1, ls)), _full((1, ls)),
           _full((ls, 19)), _full((1, 19)), _full((16, lsn)),
           _full((19, lsn)), _full((1, lsn))]
    outs = [_rows(BR_E, 19), _full((1, lsn)), _full((1, lsn))]
    oshapes = [_sd((N_E, 19)), _sd((1, lsn)), _sd((1, lsn))]
    args = [h2, src, sc2, sh2, w3, _row(b3), wm1x, wm1e, _row(bm1)]
    if we is None:
        body = _p2_body
    else:
        body = _p2_body_head
        ins += [_full((19, 1)), _full((1, 1))]
        outs.append(_rows(BR_E, 1))
        oshapes.append(_sd((N_E, 1)))
        args += [we, _row(be)]
    return pl.pallas_call(
        body, grid=(N_E // BR_E,), in_specs=ins, out_specs=outs,
        out_shape=oshapes, interpret=_INTERPRET,
    )(*args)


def _p2b_body(src, ea, wm1x, wm1e, bm1, scn, shn, h_ref):
    m1 = _dot(src[...], wm1x[...]) + _dot(ea[...], wm1e[...]) + bm1[...]
    h_ref[...] = jnp.maximum(m1 * scn[...] + shn[...], 0.0)


def _edge_p2b(src, ea, wm1x, wm1e, bm1, scn, shn):
    lsn = wm1x.shape[1]
    return pl.pallas_call(
        _p2b_body,
        grid=(N_E // BR_E,),
        in_specs=[_rows(BR_E, 16), _rows(BR_E, 19), _full((16, lsn)),
                  _full((19, lsn)), _full((1, lsn)), _full((1, lsn)),
                  _full((1, lsn))],
        out_specs=_rows(BR_E, lsn),
        out_shape=_sd((N_E, lsn)),
        interpret=_INTERPRET,
    )(src, ea, wm1x, wm1e, _row(bm1), scn, shn)


_S_CB = 15
_S_CPT = _G_NCH // _NS
_S_XTRA = _G_NCH - _S_CPT * _NS
_S_RPT = N_V // _NS


def _make_sc_scatter(ls):
    nsl = ls // 16

    def body(h_hbm, col2d, zero_hbm, out_hbm, idx_v, data_v, accum, sem):
        cid = lax.axis_index("c")
        sid = lax.axis_index("s")

        for sl in range(nsl // _NC):
            c0 = (2 * sl + cid) * 16
            pltpu.sync_copy(zero_hbm.at[pl.ds(0, _S_RPT)],
                            accum.at[pl.ds(sid * _S_RPT, _S_RPT)])
            plsc.subcore_barrier()

            def batch(b, carry):
                ch0 = sid + b * (_S_CB * _NS)
                cps = [pltpu.async_copy(col2d.at[ch0 + j * _NS],
                                        idx_v.at[j], sem)
                       for j in range(_S_CB)]
                for c in cps:
                    c.wait()
                cps = [pltpu.async_copy(
                    h_hbm.at[pl.ds((ch0 + j * _NS) * 128, 128),
                             pl.ds(c0, 16)],
                    data_v.at[pl.ds(j * 128, 128)], sem)
                    for j in range(_S_CB)]
                for c in cps:
                    c.wait()
                for j in range(_S_CB):
                    pltpu.sync_copy(data_v.at[pl.ds(j * 128, 128)],
                                    accum.at[idx_v.at[j]], add=True)
                return carry

            lax.fori_loop(0, _S_CPT // _S_CB, batch, 0)

            @pl.when(sid < _S_XTRA)
            def _():
                ch = sid + _S_CPT * _NS
                pltpu.sync_copy(col2d.at[ch], idx_v.at[0])
                pltpu.async_copy(
                    h_hbm.at[pl.ds(ch * 128, 128), pl.ds(c0, 16)],
                    data_v.at[pl.ds(0, 128)], sem).wait()
                pltpu.sync_copy(data_v.at[pl.ds(0, 128)],
                                accum.at[idx_v.at[0]], add=True)

            plsc.subcore_barrier()
            pltpu.sync_copy(
                accum.at[pl.ds(sid * _S_RPT, _S_RPT)],
                out_hbm.at[pl.ds(sid * _S_RPT, _S_RPT), pl.ds(c0, 16)])
            plsc.subcore_barrier()

    return body


def _sc_scatter(h, col2d, zeros):
    ls = h.shape[1]
    mesh = plsc.VectorSubcoreMesh(core_axis_name="c", subcore_axis_name="s")
    f = pl.kernel(
        _make_sc_scatter(ls), mesh=mesh,
        out_type=_sd((N_V, ls)),
        scratch_types=[pltpu.VMEM((_S_CB, 128), jnp.int32),
                       pltpu.VMEM((_S_CB * 128, 16), f32),
                       pltpu.VMEM_SHARED((N_V, 16), f32),
                       pltpu.SemaphoreType.DMA],
        compiler_params=pltpu.CompilerParams(use_tc_tiling_on_sc=False),
    )
    return f(h, col2d, zeros)


def _sc_counts_body(col2d, ones_hbm, zero_hbm, out_hbm,
                    idx_v, ones_v, accum, sem):
    cid = lax.axis_index("c")
    sid = lax.axis_index("s")

    @pl.when(cid == 0)
    def _():
        pltpu.sync_copy(ones_hbm, ones_v)
        pltpu.sync_copy(zero_hbm.at[pl.ds(0, _S_RPT)],
                        accum.at[pl.ds(sid * _S_RPT, _S_RPT)])
        plsc.subcore_barrier()

        def batch(b, carry):
            ch0 = sid + b * (_S_CB * _NS)
            cps = [pltpu.async_copy(col2d.at[ch0 + j * _NS], idx_v.at[j],
                                    sem)
                   for j in range(_S_CB)]
            for c in cps:
                c.wait()
            for j in range(_S_CB):
                pltpu.sync_copy(ones_v, accum.at[idx_v.at[j]], add=True)
            return carry

        lax.fori_loop(0, _S_CPT // _S_CB, batch, 0)

        @pl.when(sid < _S_XTRA)
        def _():
            ch = sid + _S_CPT * _NS
            pltpu.sync_copy(col2d.at[ch], idx_v.at[0])
            pltpu.sync_copy(ones_v, accum.at[idx_v.at[0]], add=True)

        plsc.subcore_barrier()
        pltpu.sync_copy(accum.at[pl.ds(sid * _S_RPT, _S_RPT)],
                        out_hbm.at[pl.ds(sid * _S_RPT, _S_RPT)])


def _sc_counts(col2d, ones, zeros):
    mesh = plsc.VectorSubcoreMesh(core_axis_name="c", subcore_axis_name="s")
    f = pl.kernel(
        _sc_counts_body, mesh=mesh,
        out_type=_sd((N_V, 16)),
        scratch_types=[pltpu.VMEM((_S_CB, 128), jnp.int32),
                       pltpu.VMEM((128, 16), f32),
                       pltpu.VMEM_SHARED((N_V, 16), f32),
                       pltpu.SemaphoreType.DMA],
        compiler_params=pltpu.CompilerParams(use_tc_tiling_on_sc=False),
    )
    return f(col2d, ones, zeros)


def _p4_body(x, seg, cnt, wm2x, wm2s, bm2, m2_ref, s_ref, ss_ref):
    c = jnp.maximum(cnt[...][:, 0:1], 1.0)
    segm = seg[...] * (1.0 / c)
    m2 = _dot(x[...], wm2x[...]) + _dot(segm, wm2s[...]) + bm2[...]
    m2_ref[...] = m2

    @pl.when(pl.program_id(0) == 0)
    def _():
        s_ref[...] = jnp.zeros_like(s_ref)
        ss_ref[...] = jnp.zeros_like(ss_ref)

    s_ref[...] += jnp.sum(m2, axis=0, keepdims=True)
    ss_ref[...] += jnp.sum(m2 * m2, axis=0, keepdims=True)


def _node_p4(x, seg, cnt16, wm2x, wm2s, bm2):
    ls = seg.shape[1]
    e2 = wm2x.shape[1]
    return pl.pallas_call(
        _p4_body,
        grid=(N_V // BR_N,),
        in_specs=[_rows(BR_N, 16), _rows(BR_N, ls), _rows(BR_N, 16),
                  _full((16, e2)), _full((ls, e2)), _full((1, e2))],
        out_specs=[_rows(BR_N, e2), _full((1, e2)), _full((1, e2))],
        out_shape=[_sd((N_V, e2)), _sd((1, e2)), _sd((1, e2))],
        interpret=_INTERPRET,
    )(x, seg, cnt16, wm2x, wm2s, _row(bm2))


def _p5_body_bsum(m2, oh, sc, sh, wlin, blin, x_ref, bs_ref, bc_ref):
    a = jnp.maximum(m2[...] * sc[...] + sh[...], 0.0)
    xn = jnp.maximum(_dot(a, wlin[...]) + blin[...], 0.0)
    x_ref[...] = xn

    @pl.when(pl.program_id(0) == 0)
    def _():
        bs_ref[...] = jnp.zeros_like(bs_ref)
        bc_ref[...] = jnp.zeros_like(bc_ref)

    o = oh[...]
    bs_ref[...] += lax.dot_general(o, xn, (((0,), (0,)), ((), ())),
                                   preferred_element_type=f32)
    ones = jnp.ones((o.shape[0], 1), f32)
    bc_ref[...] += lax.dot_general(o, ones, (((0,), (0,)), ((), ())),
                                   preferred_element_type=f32)


def _p5_body_head(m2, sc, sh, wlin, blin, wx, bx, x_ref, y_ref):
    a = jnp.maximum(m2[...] * sc[...] + sh[...], 0.0)
    xn = jnp.maximum(_dot(a, wlin[...]) + blin[...], 0.0)
    x_ref[...] = xn
    y_ref[...] = jax.nn.sigmoid(_dot(xn, wx[...]) + bx[...])


def _node_p5(m2, sc, sh, wlin, blin, onehot=None, wx=None, bx=None):
    e2 = m2.shape[1]
    if onehot is not None:
        return pl.pallas_call(
            _p5_body_bsum,
            grid=(N_V // BR_N,),
            in_specs=[_rows(BR_N, e2), _rows(BR_N, NG), _full((1, e2)),
                      _full((1, e2)), _full((e2, 16)), _full((1, 16))],
            out_specs=[_rows(BR_N, 16), _full((NG, 16)), _full((NG, 1))],
            out_shape=[_sd((N_V, 16)), _sd((NG, 16)), _sd((NG, 1))],
            interpret=_INTERPRET,
        )(m2, onehot, sc, sh, wlin, _row(blin))
    return pl.pallas_call(
        _p5_body_head,
        grid=(N_V // BR_N,),
        in_specs=[_rows(BR_N, e2), _full((1, e2)), _full((1, e2)),
                  _full((e2, 16)), _full((1, 16)), _full((16, 1)),
                  _full((1, 1))],
        out_specs=[_rows(BR_N, 16), _rows(BR_N, 1)],
        out_shape=[_sd((N_V, 16)), _sd((N_V, 1))],
        interpret=_INTERPRET,
    )(m2, sc, sh, wlin, _row(blin), wx, _row(bx))


def _p6_body(u, bs, bc, w1u, w1s, b1, g1, bb1, w2, b2, g2, bb2, w3, b3,
             out_ref):
    cnt = jnp.maximum(bc[...], 1.0)
    seg = bs[...] * (1.0 / cnt)
    h = _dot(u[...], w1u[...]) + _dot(seg, w1s[...]) + b1[...]

    def _bn(h, g, bb):
        mu = jnp.mean(h, axis=0, keepdims=True)
        var = jnp.mean(h * h, axis=0, keepdims=True) - mu * mu
        return (h - mu) / jnp.sqrt(var + 1e-5) * g[...] + bb[...]

    a1 = jnp.maximum(_bn(h, g1, bb1), 0.0)
    h2 = _dot(a1, w2[...]) + b2[...]
    a2 = jnp.maximum(_bn(h2, g2, bb2), 0.0)
    out_ref[...] = _dot(a2, w3[...]) + b3[...]


def _glob_p6(u, bs, bc, gp):
    w1u, w1s = gp["l1"]["w"][:7], gp["l1"]["w"][7:]
    return pl.pallas_call(
        _p6_body,
        in_specs=[_full((NG, 7)), _full((NG, 16)), _full((NG, 1)),
                  _full((7, 128)), _full((16, 128)), _full((1, 128)),
                  _full((1, 128)), _full((1, 128)), _full((128, 128)),
                  _full((1, 128)), _full((1, 128)), _full((1, 128)),
                  _full((128, 7)), _full((1, 7))],
        out_specs=_full((NG, 7)),
        out_shape=_sd((NG, 7)),
        interpret=_INTERPRET,
    )(u, bs, bc, w1u, w1s, _row(gp["l1"]["b"]), _row(gp["bn1"]["g"]),
      _row(gp["bn1"]["b"]), gp["l2"]["w"], _row(gp["l2"]["b"]),
      _row(gp["bn2"]["g"]), _row(gp["bn2"]["b"]), gp["l3"]["w"],
      _row(gp["l3"]["b"]))


_G_CPW = _G_NCH // _NW
_G_XTRA = _G_NCH - _G_CPW * _NW
_G_CB = 15
_G_NB = _G_CPW // _G_CB


def _sc_gather_body(x_hbm, row2d, col2d, src_hbm, dst_hbm,
                    idx_v, rows_v, sem):
    wid = lax.axis_index("s") * _NC + lax.axis_index("c")

    def run(ih2d, oh):
        def batch(b, carry):
            c0 = wid + b * (_G_CB * _NW)
            cps = [pltpu.async_copy(ih2d.at[c0 + j * _NW], idx_v.at[j], sem)
                   for j in range(_G_CB)]
            for c in cps:
                c.wait()
            cps = [pltpu.async_copy(x_hbm.at[idx_v.at[j]],
                                    rows_v.at[pl.ds(j * 128, 128)], sem)
                   for j in range(_G_CB)]
            for c in cps:
                c.wait()
            cps = [pltpu.async_copy(rows_v.at[pl.ds(j * 128, 128)],
                                    oh.at[pl.ds((c0 + j * _NW) * 128, 128)],
                                    sem)
                   for j in range(_G_CB)]
            for c in cps:
                c.wait()
            return carry

        lax.fori_loop(0, _G_NB, batch, 0)

        @pl.when(wid < _G_XTRA)
        def _():
            c = wid + _G_CPW * _NW
            pltpu.sync_copy(ih2d.at[c], idx_v.at[0])
            pltpu.async_copy(x_hbm.at[idx_v.at[0]],
                             rows_v.at[pl.ds(0, 128)], sem).wait()
            pltpu.sync_copy(rows_v.at[pl.ds(0, 128)],
                            oh.at[pl.ds(c * 128, 128)])

    run(row2d, src_hbm)
    run(col2d, dst_hbm)


def _sc_gather(x, row2d, col2d):
    mesh = plsc.VectorSubcoreMesh(core_axis_name="c", subcore_axis_name="s")
    f = pl.kernel(
        _sc_gather_body, mesh=mesh,
        out_type=[_sd((N_E, 16)), _sd((N_E, 16))],
        scratch_types=[pltpu.VMEM((_G_CB, 128), jnp.int32),
                       pltpu.VMEM((_G_CB * 128, 16), f32),
                       pltpu.SemaphoreType.DMA],
        compiler_params=pltpu.CompilerParams(use_tc_tiling_on_sc=False),
    )
    return f(x, row2d, col2d)


def kernel(x, edge_index, edge_attr, u, batch, params):
    row, col = edge_index[0], edge_index[1]
    row2d = row.reshape(N_E // 128, 128)
    col2d = col.reshape(N_E // 128, 128)
    onehot = (batch[:, None] == jnp.arange(NG, dtype=jnp.int32)[None, :]
              ).astype(f32)
    zeros = jnp.zeros((_S_RPT, 16), f32)
    ones = jnp.ones((128, 16), f32)
    cnt16 = _sc_counts(col2d, ones, zeros)
    ea = edge_attr
    y_pred = None
    head = None
    for li, name in enumerate(("ml1", "ml2", "ml3")):
        p = params[name]
        ep, np_, gp = p["edge"], p["node"], p["glob"]
        last = li == 2
        src, dst = _sc_gather(x, row2d, col2d)
        w1 = ep["l1"]["w"]
        w1s, w1d, w1e = w1[:16], w1[16:32], w1[32:]
        s1, ss1 = _edge_stats1(src, dst, ea, w1s, w1d, w1e, ep["l1"]["b"])
        sc1, sh1 = _fin_stats(s1[0], ss1[0], N_E, ep["bn1"]["g"],
                              ep["bn1"]["b"])
        h2, s2, ss2 = _edge_p1(src, dst, ea, w1s, w1d, w1e, ep["l1"]["b"],
                               sc1, sh1, ep["l2"]["w"], ep["l2"]["b"])
        sc2, sh2 = _fin_stats(s2[0], ss2[0], N_E, ep["bn2"]["g"],
                              ep["bn2"]["b"])
        wm1 = np_["m1"]["w"]
        wm1x, wm1e = wm1[:16], wm1[16:]
        if last:
            ea, sn, ssn, head = _edge_p2(
                h2, src, sc2, sh2, ep["lin"]["w"], ep["lin"]["b"],
                wm1x, wm1e, np_["m1"]["b"],
                params["e_lin"]["w"], params["e_lin"]["b"])
        else:
            ea, sn, ssn = _edge_p2(
                h2, src, sc2, sh2, ep["lin"]["w"], ep["lin"]["b"],
                wm1x, wm1e, np_["m1"]["b"])
        scn, shn = _fin_stats(sn[0], ssn[0], N_E, np_["bn1"]["g"],
                              np_["bn1"]["b"])
        h = _edge_p2b(src, ea, wm1x, wm1e, np_["m1"]["b"], scn, shn)
        seg = _sc_scatter(h, col2d, zeros)
        wm2 = np_["m2"]["w"]
        m2, sm, ssm = _node_p4(x, seg, cnt16, wm2[:16], wm2[16:],
                               np_["m2"]["b"])
        scm, shm = _fin_stats(sm[0], ssm[0], N_V, np_["bn2"]["g"],
                              np_["bn2"]["b"])
        if last:
            x, y_pred = _node_p5(m2, scm, shm, np_["lin"]["w"],
                                 np_["lin"]["b"], wx=params["x_lin"]["w"],
                                 bx=params["x_lin"]["b"])
        else:
            x, bs, bc = _node_p5(m2, scm, shm, np_["lin"]["w"],
                                 np_["lin"]["b"], onehot=onehot)
            u = _glob_p6(u, bs, bc, gp)
    return (y_pred, head)


# speedup vs baseline: 2.1786x; 1.1654x over previous
import functools

import jax
import jax.numpy as jnp
from jax import lax
from jax.experimental import pallas as pl
from jax.experimental.pallas import tpu as pltpu
from jax.experimental.pallas import tpu_sc as plsc

_INTERPRET = False

_NC = 2
_NS = 16
_NW = _NC * _NS
_G_NCH = 800000 // 128

N_E = 800000
N_V = 50000
NG = 64
BR_E = 4000
BR_N = 2000
f32 = jnp.float32


def _row(v):
    return v.reshape(1, -1)


def _fin_stats(s, ss, n, g, bbn):
    mu = s / n
    var = ss / n - mu * mu
    isd = 1.0 / jnp.sqrt(var + 1e-5)
    sc = g * isd
    sh = bbn - mu * sc
    return _row(sc), _row(sh)


def _dot(a, b):
    return jnp.dot(a, b, preferred_element_type=f32)


def _full(shape):
    return pl.BlockSpec(shape, lambda *a: tuple(0 for _ in shape))


def _rows(br, w):
    return pl.BlockSpec((br, w), lambda i: (i, 0))


def _sd(shape):
    return jax.ShapeDtypeStruct(shape, f32)


def _p0_body(src, dst, ea, w1s, w1d, w1e, b1, s_ref, ss_ref):
    h = (_dot(src[...], w1s[...]) + _dot(dst[...], w1d[...])
         + _dot(ea[...], w1e[...]) + b1[...])

    @pl.when(pl.program_id(0) == 0)
    def _():
        s_ref[...] = jnp.zeros_like(s_ref)
        ss_ref[...] = jnp.zeros_like(ss_ref)

    s_ref[...] += jnp.sum(h, axis=0, keepdims=True)
    ss_ref[...] += jnp.sum(h * h, axis=0, keepdims=True)


def _edge_stats1(src, dst, ea, w1s, w1d, w1e, b1):
    ls = w1s.shape[1]
    s, ss = pl.pallas_call(
        _p0_body,
        grid=(N_E // BR_E,),
        in_specs=[_rows(BR_E, 16), _rows(BR_E, 16), _rows(BR_E, 19),
                  _full((16, ls)), _full((16, ls)), _full((19, ls)),
                  _full((1, ls))],
        out_specs=[_full((1, ls)), _full((1, ls))],
        out_shape=[_sd((1, ls))] * 2,
        interpret=_INTERPRET,
    )(src, dst, ea, w1s, w1d, w1e, _row(b1))
    return s, ss


def _p1_body(src, dst, ea, w1s, w1d, w1e, b1, sc1, sh1, w2, b2,
             h2_ref, s_ref, ss_ref):
    h1 = (_dot(src[...], w1s[...]) + _dot(dst[...], w1d[...])
          + _dot(ea[...], w1e[...]) + b1[...])
    a1 = jnp.maximum(h1 * sc1[...] + sh1[...], 0.0)
    h2 = _dot(a1, w2[...]) + b2[...]
    h2_ref[...] = h2

    @pl.when(pl.program_id(0) == 0)
    def _():
        s_ref[...] = jnp.zeros_like(s_ref)
        ss_ref[...] = jnp.zeros_like(ss_ref)

    s_ref[...] += jnp.sum(h2, axis=0, keepdims=True)
    ss_ref[...] += jnp.sum(h2 * h2, axis=0, keepdims=True)


def _edge_p1(src, dst, ea, w1s, w1d, w1e, b1, sc1, sh1, w2, b2):
    ls = w2.shape[1]
    return pl.pallas_call(
        _p1_body,
        grid=(N_E // BR_E,),
        in_specs=[_rows(BR_E, 16), _rows(BR_E, 16), _rows(BR_E, 19),
                  _full((16, ls)), _full((16, ls)), _full((19, ls)),
                  _full((1, ls)), _full((1, ls)), _full((1, ls)),
                  _full((ls, ls)), _full((1, ls))],
        out_specs=[_rows(BR_E, ls), _full((1, ls)), _full((1, ls))],
        out_shape=[_sd((N_E, ls)), _sd((1, ls)), _sd((1, ls))],
        interpret=_INTERPRET,
    )(src, dst, ea, w1s, w1d, w1e, _row(b1), sc1, sh1, w2, _row(b2))


def _p2_body(h2, src, sc2, sh2, w3, b3, wm1x, wm1e, bm1,
             ea_ref, s_ref, ss_ref):
    a2 = jnp.maximum(h2[...] * sc2[...] + sh2[...], 0.0)
    ea = jnp.maximum(_dot(a2, w3[...]) + b3[...], 0.0)
    ea_ref[...] = ea
    m1 = _dot(src[...], wm1x[...]) + _dot(ea, wm1e[...]) + bm1[...]

    @pl.when(pl.program_id(0) == 0)
    def _():
        s_ref[...] = jnp.zeros_like(s_ref)
        ss_ref[...] = jnp.zeros_like(ss_ref)

    s_ref[...] += jnp.sum(m1, axis=0, keepdims=True)
    ss_ref[...] += jnp.sum(m1 * m1, axis=0, keepdims=True)


def _p2_body_head(h2, src, sc2, sh2, w3, b3, wm1x, wm1e, bm1, we, be,
                  ea_ref, s_ref, ss_ref, hd_ref):
    a2 = jnp.maximum(h2[...] * sc2[...] + sh2[...], 0.0)
    ea = jnp.maximum(_dot(a2, w3[...]) + b3[...], 0.0)
    ea_ref[...] = ea
    m1 = _dot(src[...], wm1x[...]) + _dot(ea, wm1e[...]) + bm1[...]

    @pl.when(pl.program_id(0) == 0)
    def _():
        s_ref[...] = jnp.zeros_like(s_ref)
        ss_ref[...] = jnp.zeros_like(ss_ref)

    s_ref[...] += jnp.sum(m1, axis=0, keepdims=True)
    ss_ref[...] += jnp.sum(m1 * m1, axis=0, keepdims=True)
    hd_ref[...] = jax.nn.sigmoid(_dot(ea, we[...]) + be[...])


def _edge_p2(h2, src, sc2, sh2, w3, b3, wm1x, wm1e, bm1, we=None, be=None):
    ls = h2.shape[1]
    lsn = wm1x.shape[1]
    ins = [_rows(BR_E, ls), _rows(BR_E, 16), _full((1, ls)), _full((1, ls)),
           _full((ls, 19)), _full((1, 19)), _full((16, lsn)),
           _full((19, lsn)), _full((1, lsn))]
    outs = [_rows(BR_E, 19), _full((1, lsn)), _full((1, lsn))]
    oshapes = [_sd((N_E, 19)), _sd((1, lsn)), _sd((1, lsn))]
    args = [h2, src, sc2, sh2, w3, _row(b3), wm1x, wm1e, _row(bm1)]
    if we is None:
        body = _p2_body
    else:
        body = _p2_body_head
        ins += [_full((19, 1)), _full((1, 1))]
        outs.append(_rows(BR_E, 1))
        oshapes.append(_sd((N_E, 1)))
        args += [we, _row(be)]
    return pl.pallas_call(
        body, grid=(N_E // BR_E,), in_specs=ins, out_specs=outs,
        out_shape=oshapes, interpret=_INTERPRET,
    )(*args)


def _p2b_body(src, ea, wm1x, wm1e, bm1, scn, shn, h_ref):
    m1 = _dot(src[...], wm1x[...]) + _dot(ea[...], wm1e[...]) + bm1[...]
    h_ref[...] = jnp.maximum(m1 * scn[...] + shn[...], 0.0)


def _edge_p2b(src, ea, wm1x, wm1e, bm1, scn, shn):
    lsn = wm1x.shape[1]
    return pl.pallas_call(
        _p2b_body,
        grid=(N_E // BR_E,),
        in_specs=[_rows(BR_E, 16), _rows(BR_E, 19), _full((16, lsn)),
                  _full((19, lsn)), _full((1, lsn)), _full((1, lsn)),
                  _full((1, lsn))],
        out_specs=_rows(BR_E, lsn),
        out_shape=_sd((N_E, lsn)),
        interpret=_INTERPRET,
    )(src, ea, wm1x, wm1e, _row(bm1), scn, shn)


_S_CB = 15
_S_CPT = _G_NCH // _NS
_S_XTRA = _G_NCH - _S_CPT * _NS
_S_RPT = N_V // _NS


def _make_sc_scatter(ls):
    nsl = ls // 16

    def body(h_hbm, col2d, zero_hbm, out_hbm, idx_v, data_v, accum, sem):
        cid = lax.axis_index("c")
        sid = lax.axis_index("s")

        for sl in range(nsl // _NC):
            c0 = (2 * sl + cid) * 16
            pltpu.sync_copy(zero_hbm.at[pl.ds(0, _S_RPT)],
                            accum.at[pl.ds(sid * _S_RPT, _S_RPT)])
            plsc.subcore_barrier()

            def batch(b, carry):
                ch0 = sid + b * (_S_CB * _NS)
                cps = [pltpu.async_copy(col2d.at[ch0 + j * _NS],
                                        idx_v.at[j], sem)
                       for j in range(_S_CB)]
                for c in cps:
                    c.wait()
                cps = [pltpu.async_copy(
                    h_hbm.at[pl.ds((ch0 + j * _NS) * 128, 128),
                             pl.ds(c0, 16)],
                    data_v.at[pl.ds(j * 128, 128)], sem)
                    for j in range(_S_CB)]
                for c in cps:
                    c.wait()
                cps = [pltpu.async_copy(data_v.at[pl.ds(j * 128, 128)],
                                        accum.at[idx_v.at[j]], sem,
                                        add=True)
                       for j in range(_S_CB)]
                for c in cps:
                    c.wait()
                return carry

            lax.fori_loop(0, _S_CPT // _S_CB, batch, 0)

            @pl.when(sid < _S_XTRA)
            def _():
                ch = sid + _S_CPT * _NS
                pltpu.sync_copy(col2d.at[ch], idx_v.at[0])
                pltpu.async_copy(
                    h_hbm.at[pl.ds(ch * 128, 128), pl.ds(c0, 16)],
                    data_v.at[pl.ds(0, 128)], sem).wait()
                pltpu.sync_copy(data_v.at[pl.ds(0, 128)],
                                accum.at[idx_v.at[0]], add=True)

            plsc.subcore_barrier()
            pltpu.sync_copy(
                accum.at[pl.ds(sid * _S_RPT, _S_RPT)],
                out_hbm.at[pl.ds(sid * _S_RPT, _S_RPT), pl.ds(c0, 16)])
            plsc.subcore_barrier()

    return body


def _sc_scatter(h, col2d, zeros):
    ls = h.shape[1]
    mesh = plsc.VectorSubcoreMesh(core_axis_name="c", subcore_axis_name="s")
    f = pl.kernel(
        _make_sc_scatter(ls), mesh=mesh,
        out_type=_sd((N_V, ls)),
        scratch_types=[pltpu.VMEM((_S_CB, 128), jnp.int32),
                       pltpu.VMEM((_S_CB * 128, 16), f32),
                       pltpu.VMEM_SHARED((N_V, 16), f32),
                       pltpu.SemaphoreType.DMA],
        compiler_params=pltpu.CompilerParams(use_tc_tiling_on_sc=False),
    )
    return f(h, col2d, zeros)


def _sc_counts_body(col2d, ones_hbm, zero_hbm, out_hbm,
                    idx_v, ones_v, accum, sem):
    cid = lax.axis_index("c")
    sid = lax.axis_index("s")

    @pl.when(cid == 0)
    def _():
        pltpu.sync_copy(ones_hbm, ones_v)
        pltpu.sync_copy(zero_hbm.at[pl.ds(0, _S_RPT)],
                        accum.at[pl.ds(sid * _S_RPT, _S_RPT)])
        plsc.subcore_barrier()

        def batch(b, carry):
            ch0 = sid + b * (_S_CB * _NS)
            cps = [pltpu.async_copy(col2d.at[ch0 + j * _NS], idx_v.at[j],
                                    sem)
                   for j in range(_S_CB)]
            for c in cps:
                c.wait()
            cps = [pltpu.async_copy(ones_v, accum.at[idx_v.at[j]], sem,
                                    add=True)
                   for j in range(_S_CB)]
            for c in cps:
                c.wait()
            return carry

        lax.fori_loop(0, _S_CPT // _S_CB, batch, 0)

        @pl.when(sid < _S_XTRA)
        def _():
            ch = sid + _S_CPT * _NS
            pltpu.sync_copy(col2d.at[ch], idx_v.at[0])
            pltpu.sync_copy(ones_v, accum.at[idx_v.at[0]], add=True)

        plsc.subcore_barrier()
        pltpu.sync_copy(accum.at[pl.ds(sid * _S_RPT, _S_RPT)],
                        out_hbm.at[pl.ds(sid * _S_RPT, _S_RPT)])


def _sc_counts(col2d, ones, zeros):
    mesh = plsc.VectorSubcoreMesh(core_axis_name="c", subcore_axis_name="s")
    f = pl.kernel(
        _sc_counts_body, mesh=mesh,
        out_type=_sd((N_V, 16)),
        scratch_types=[pltpu.VMEM((_S_CB, 128), jnp.int32),
                       pltpu.VMEM((128, 16), f32),
                       pltpu.VMEM_SHARED((N_V, 16), f32),
                       pltpu.SemaphoreType.DMA],
        compiler_params=pltpu.CompilerParams(use_tc_tiling_on_sc=False),
    )
    return f(col2d, ones, zeros)


def _p4_body(x, seg, cnt, wm2x, wm2s, bm2, m2_ref, s_ref, ss_ref):
    c = jnp.maximum(cnt[...][:, 0:1], 1.0)
    segm = seg[...] * (1.0 / c)
    m2 = _dot(x[...], wm2x[...]) + _dot(segm, wm2s[...]) + bm2[...]
    m2_ref[...] = m2

    @pl.when(pl.program_id(0) == 0)
    def _():
        s_ref[...] = jnp.zeros_like(s_ref)
        ss_ref[...] = jnp.zeros_like(ss_ref)

    s_ref[...] += jnp.sum(m2, axis=0, keepdims=True)
    ss_ref[...] += jnp.sum(m2 * m2, axis=0, keepdims=True)


def _node_p4(x, seg, cnt16, wm2x, wm2s, bm2):
    ls = seg.shape[1]
    e2 = wm2x.shape[1]
    return pl.pallas_call(
        _p4_body,
        grid=(N_V // BR_N,),
        in_specs=[_rows(BR_N, 16), _rows(BR_N, ls), _rows(BR_N, 16),
                  _full((16, e2)), _full((ls, e2)), _full((1, e2))],
        out_specs=[_rows(BR_N, e2), _full((1, e2)), _full((1, e2))],
        out_shape=[_sd((N_V, e2)), _sd((1, e2)), _sd((1, e2))],
        interpret=_INTERPRET,
    )(x, seg, cnt16, wm2x, wm2s, _row(bm2))


def _p5_body_bsum(m2, oh, sc, sh, wlin, blin, x_ref, bs_ref, bc_ref):
    a = jnp.maximum(m2[...] * sc[...] + sh[...], 0.0)
    xn = jnp.maximum(_dot(a, wlin[...]) + blin[...], 0.0)
    x_ref[...] = xn

    @pl.when(pl.program_id(0) == 0)
    def _():
        bs_ref[...] = jnp.zeros_like(bs_ref)
        bc_ref[...] = jnp.zeros_like(bc_ref)

    o = oh[...]
    bs_ref[...] += lax.dot_general(o, xn, (((0,), (0,)), ((), ())),
                                   preferred_element_type=f32)
    ones = jnp.ones((o.shape[0], 1), f32)
    bc_ref[...] += lax.dot_general(o, ones, (((0,), (0,)), ((), ())),
                                   preferred_element_type=f32)


def _p5_body_head(m2, sc, sh, wlin, blin, wx, bx, x_ref, y_ref):
    a = jnp.maximum(m2[...] * sc[...] + sh[...], 0.0)
    xn = jnp.maximum(_dot(a, wlin[...]) + blin[...], 0.0)
    x_ref[...] = xn
    y_ref[...] = jax.nn.sigmoid(_dot(xn, wx[...]) + bx[...])


def _node_p5(m2, sc, sh, wlin, blin, onehot=None, wx=None, bx=None):
    e2 = m2.shape[1]
    if onehot is not None:
        return pl.pallas_call(
            _p5_body_bsum,
            grid=(N_V // BR_N,),
            in_specs=[_rows(BR_N, e2), _rows(BR_N, NG), _full((1, e2)),
                      _full((1, e2)), _full((e2, 16)), _full((1, 16))],
            out_specs=[_rows(BR_N, 16), _full((NG, 16)), _full((NG, 1))],
            out_shape=[_sd((N_V, 16)), _sd((NG, 16)), _sd((NG, 1))],
            interpret=_INTERPRET,
        )(m2, onehot, sc, sh, wlin, _row(blin))
    return pl.pallas_call(
        _p5_body_head,
        grid=(N_V // BR_N,),
        in_specs=[_rows(BR_N, e2), _full((1, e2)), _full((1, e2)),
                  _full((e2, 16)), _full((1, 16)), _full((16, 1)),
                  _full((1, 1))],
        out_specs=[_rows(BR_N, 16), _rows(BR_N, 1)],
        out_shape=[_sd((N_V, 16)), _sd((N_V, 1))],
        interpret=_INTERPRET,
    )(m2, sc, sh, wlin, _row(blin), wx, _row(bx))


def _p6_body(u, bs, bc, w1u, w1s, b1, g1, bb1, w2, b2, g2, bb2, w3, b3,
             out_ref):
    cnt = jnp.maximum(bc[...], 1.0)
    seg = bs[...] * (1.0 / cnt)
    h = _dot(u[...], w1u[...]) + _dot(seg, w1s[...]) + b1[...]

    def _bn(h, g, bb):
        mu = jnp.mean(h, axis=0, keepdims=True)
        var = jnp.mean(h * h, axis=0, keepdims=True) - mu * mu
        return (h - mu) / jnp.sqrt(var + 1e-5) * g[...] + bb[...]

    a1 = jnp.maximum(_bn(h, g1, bb1), 0.0)
    h2 = _dot(a1, w2[...]) + b2[...]
    a2 = jnp.maximum(_bn(h2, g2, bb2), 0.0)
    out_ref[...] = _dot(a2, w3[...]) + b3[...]


def _glob_p6(u, bs, bc, gp):
    w1u, w1s = gp["l1"]["w"][:7], gp["l1"]["w"][7:]
    return pl.pallas_call(
        _p6_body,
        in_specs=[_full((NG, 7)), _full((NG, 16)), _full((NG, 1)),
                  _full((7, 128)), _full((16, 128)), _full((1, 128)),
                  _full((1, 128)), _full((1, 128)), _full((128, 128)),
                  _full((1, 128)), _full((1, 128)), _full((1, 128)),
                  _full((128, 7)), _full((1, 7))],
        out_specs=_full((NG, 7)),
        out_shape=_sd((NG, 7)),
        interpret=_INTERPRET,
    )(u, bs, bc, w1u, w1s, _row(gp["l1"]["b"]), _row(gp["bn1"]["g"]),
      _row(gp["bn1"]["b"]), gp["l2"]["w"], _row(gp["l2"]["b"]),
      _row(gp["bn2"]["g"]), _row(gp["bn2"]["b"]), gp["l3"]["w"],
      _row(gp["l3"]["b"]))


_G_CPW = _G_NCH // _NW
_G_XTRA = _G_NCH - _G_CPW * _NW
_G_CB = 15
_G_NB = _G_CPW // _G_CB


def _sc_gather_body(x_hbm, row2d, col2d, src_hbm, dst_hbm,
                    idx_v, rows_v, sem):
    wid = lax.axis_index("s") * _NC + lax.axis_index("c")

    def run(ih2d, oh):
        def batch(b, carry):
            c0 = wid + b * (_G_CB * _NW)
            cps = [pltpu.async_copy(ih2d.at[c0 + j * _NW], idx_v.at[j], sem)
                   for j in range(_G_CB)]
            for c in cps:
                c.wait()
            cps = [pltpu.async_copy(x_hbm.at[idx_v.at[j]],
                                    rows_v.at[pl.ds(j * 128, 128)], sem)
                   for j in range(_G_CB)]
            for c in cps:
                c.wait()
            cps = [pltpu.async_copy(rows_v.at[pl.ds(j * 128, 128)],
                                    oh.at[pl.ds((c0 + j * _NW) * 128, 128)],
                                    sem)
                   for j in range(_G_CB)]
            for c in cps:
                c.wait()
            return carry

        lax.fori_loop(0, _G_NB, batch, 0)

        @pl.when(wid < _G_XTRA)
        def _():
            c = wid + _G_CPW * _NW
            pltpu.sync_copy(ih2d.at[c], idx_v.at[0])
            pltpu.async_copy(x_hbm.at[idx_v.at[0]],
                             rows_v.at[pl.ds(0, 128)], sem).wait()
            pltpu.sync_copy(rows_v.at[pl.ds(0, 128)],
                            oh.at[pl.ds(c * 128, 128)])

    run(row2d, src_hbm)
    run(col2d, dst_hbm)


def _sc_gather(x, row2d, col2d):
    mesh = plsc.VectorSubcoreMesh(core_axis_name="c", subcore_axis_name="s")
    f = pl.kernel(
        _sc_gather_body, mesh=mesh,
        out_type=[_sd((N_E, 16)), _sd((N_E, 16))],
        scratch_types=[pltpu.VMEM((_G_CB, 128), jnp.int32),
                       pltpu.VMEM((_G_CB * 128, 16), f32),
                       pltpu.SemaphoreType.DMA],
        compiler_params=pltpu.CompilerParams(use_tc_tiling_on_sc=False),
    )
    return f(x, row2d, col2d)


def kernel(x, edge_index, edge_attr, u, batch, params):
    row, col = edge_index[0], edge_index[1]
    row2d = row.reshape(N_E // 128, 128)
    col2d = col.reshape(N_E // 128, 128)
    onehot = (batch[:, None] == jnp.arange(NG, dtype=jnp.int32)[None, :]
              ).astype(f32)
    zeros = jnp.zeros((_S_RPT, 16), f32)
    ones = jnp.ones((128, 16), f32)
    cnt16 = _sc_counts(col2d, ones, zeros)
    ea = edge_attr
    y_pred = None
    head = None
    for li, name in enumerate(("ml1", "ml2", "ml3")):
        p = params[name]
        ep, np_, gp = p["edge"], p["node"], p["glob"]
        last = li == 2
        src, dst = _sc_gather(x, row2d, col2d)
        w1 = ep["l1"]["w"]
        w1s, w1d, w1e = w1[:16], w1[16:32], w1[32:]
        s1, ss1 = _edge_stats1(src, dst, ea, w1s, w1d, w1e, ep["l1"]["b"])
        sc1, sh1 = _fin_stats(s1[0], ss1[0], N_E, ep["bn1"]["g"],
                              ep["bn1"]["b"])
        h2, s2, ss2 = _edge_p1(src, dst, ea, w1s, w1d, w1e, ep["l1"]["b"],
                               sc1, sh1, ep["l2"]["w"], ep["l2"]["b"])
        sc2, sh2 = _fin_stats(s2[0], ss2[0], N_E, ep["bn2"]["g"],
                              ep["bn2"]["b"])
        wm1 = np_["m1"]["w"]
        wm1x, wm1e = wm1[:16], wm1[16:]
        if last:
            ea, sn, ssn, head = _edge_p2(
                h2, src, sc2, sh2, ep["lin"]["w"], ep["lin"]["b"],
                wm1x, wm1e, np_["m1"]["b"],
                params["e_lin"]["w"], params["e_lin"]["b"])
        else:
            ea, sn, ssn = _edge_p2(
                h2, src, sc2, sh2, ep["lin"]["w"], ep["lin"]["b"],
                wm1x, wm1e, np_["m1"]["b"])
        scn, shn = _fin_stats(sn[0], ssn[0], N_E, np_["bn1"]["g"],
                              np_["bn1"]["b"])
        h = _edge_p2b(src, ea, wm1x, wm1e, np_["m1"]["b"], scn, shn)
        seg = _sc_scatter(h, col2d, zeros)
        wm2 = np_["m2"]["w"]
        m2, sm, ssm = _node_p4(x, seg, cnt16, wm2[:16], wm2[16:],
                               np_["m2"]["b"])
        scm, shm = _fin_stats(sm[0], ssm[0], N_V, np_["bn2"]["g"],
                              np_["bn2"]["b"])
        if last:
            x, y_pred = _node_p5(m2, scm, shm, np_["lin"]["w"],
                                 np_["lin"]["b"], wx=params["x_lin"]["w"],
                                 bx=params["x_lin"]["b"])
        else:
            x, bs, bc = _node_p5(m2, scm, shm, np_["lin"]["w"],
                                 np_["lin"]["b"], onehot=onehot)
            u = _glob_p6(u, bs, bc, gp)
    return (y_pred, head)


# speedup vs baseline: 2.2298x; 1.0235x over previous
import functools

import jax
import jax.numpy as jnp
from jax import lax
from jax.experimental import pallas as pl
from jax.experimental.pallas import tpu as pltpu
from jax.experimental.pallas import tpu_sc as plsc

_INTERPRET = False

_NC = 2
_NS = 16
_NW = _NC * _NS
_G_NCH = 800000 // 128

N_E = 800000
N_V = 50000
NG = 64
BR_E = 4000
BR_N = 2000
f32 = jnp.float32
bf16 = jnp.bfloat16


def _row(v):
    return v.reshape(1, -1)


def _fin_stats(s, ss, n, g, bbn):
    mu = s / n
    var = ss / n - mu * mu
    isd = 1.0 / jnp.sqrt(var + 1e-5)
    sc = g * isd
    sh = bbn - mu * sc
    return _row(sc), _row(sh)


def _dot(a, b):
    return jnp.dot(a, b, preferred_element_type=f32)


def _dotb(a, b):
    return jnp.dot(a.astype(bf16), b, preferred_element_type=f32)


def _full(shape):
    return pl.BlockSpec(shape, lambda *a: tuple(0 for _ in shape))


def _rows(br, w):
    return pl.BlockSpec((br, w), lambda i: (i, 0))


def _sd(shape):
    return jax.ShapeDtypeStruct(shape, f32)


def _p0_body(src, dst, ea, w1s, w1d, w1e, b1, s_ref, ss_ref):
    h = (_dotb(src[...], w1s[...]) + _dotb(dst[...], w1d[...])
         + _dotb(ea[...], w1e[...]) + b1[...])

    @pl.when(pl.program_id(0) == 0)
    def _():
        s_ref[...] = jnp.zeros_like(s_ref)
        ss_ref[...] = jnp.zeros_like(ss_ref)

    s_ref[...] += jnp.sum(h, axis=0, keepdims=True)
    ss_ref[...] += jnp.sum(h * h, axis=0, keepdims=True)


def _edge_stats1(src, dst, ea, w1s, w1d, w1e, b1):
    ls = w1s.shape[1]
    s, ss = pl.pallas_call(
        _p0_body,
        grid=(N_E // BR_E,),
        in_specs=[_rows(BR_E, 16), _rows(BR_E, 16), _rows(BR_E, 19),
                  _full((16, ls)), _full((16, ls)), _full((19, ls)),
                  _full((1, ls))],
        out_specs=[_full((1, ls)), _full((1, ls))],
        out_shape=[_sd((1, ls))] * 2,
        interpret=_INTERPRET,
    )(src, dst, ea, w1s, w1d, w1e, _row(b1))
    return s, ss


def _p1_body(src, dst, ea, w1s, w1d, w1e, b1, sc1, sh1, w2, b2,
             h2_ref, s_ref, ss_ref):
    h1 = (_dotb(src[...], w1s[...]) + _dotb(dst[...], w1d[...])
          + _dotb(ea[...], w1e[...]) + b1[...])
    a1 = jnp.maximum(h1 * sc1[...] + sh1[...], 0.0)
    h2 = _dotb(a1, w2[...]) + b2[...]
    h2_ref[...] = h2.astype(bf16)

    @pl.when(pl.program_id(0) == 0)
    def _():
        s_ref[...] = jnp.zeros_like(s_ref)
        ss_ref[...] = jnp.zeros_like(ss_ref)

    s_ref[...] += jnp.sum(h2, axis=0, keepdims=True)
    ss_ref[...] += jnp.sum(h2 * h2, axis=0, keepdims=True)


def _edge_p1(src, dst, ea, w1s, w1d, w1e, b1, sc1, sh1, w2, b2):
    ls = w2.shape[1]
    return pl.pallas_call(
        _p1_body,
        grid=(N_E // BR_E,),
        in_specs=[_rows(BR_E, 16), _rows(BR_E, 16), _rows(BR_E, 19),
                  _full((16, ls)), _full((16, ls)), _full((19, ls)),
                  _full((1, ls)), _full((1, ls)), _full((1, ls)),
                  _full((ls, ls)), _full((1, ls))],
        out_specs=[_rows(BR_E, ls), _full((1, ls)), _full((1, ls))],
        out_shape=[jax.ShapeDtypeStruct((N_E, ls), bf16), _sd((1, ls)),
                   _sd((1, ls))],
        interpret=_INTERPRET,
    )(src, dst, ea, w1s, w1d, w1e, _row(b1), sc1, sh1, w2, _row(b2))


def _p2_body(h2, src, sc2, sh2, w3, b3, wm1x, wm1e, bm1,
             ea_ref, s_ref, ss_ref):
    a2 = jnp.maximum(h2[...].astype(f32) * sc2[...] + sh2[...], 0.0)
    ea = jnp.maximum(_dotb(a2, w3[...]) + b3[...], 0.0)
    ea_ref[...] = ea
    m1 = _dotb(src[...], wm1x[...]) + _dotb(ea, wm1e[...]) + bm1[...]

    @pl.when(pl.program_id(0) == 0)
    def _():
        s_ref[...] = jnp.zeros_like(s_ref)
        ss_ref[...] = jnp.zeros_like(ss_ref)

    s_ref[...] += jnp.sum(m1, axis=0, keepdims=True)
    ss_ref[...] += jnp.sum(m1 * m1, axis=0, keepdims=True)


def _p2_body_head(h2, src, sc2, sh2, w3, b3, wm1x, wm1e, bm1, we, be,
                  ea_ref, s_ref, ss_ref, hd_ref):
    a2 = jnp.maximum(h2[...].astype(f32) * sc2[...] + sh2[...], 0.0)
    ea = jnp.maximum(_dotb(a2, w3[...]) + b3[...], 0.0)
    ea_ref[...] = ea
    m1 = _dotb(src[...], wm1x[...]) + _dotb(ea, wm1e[...]) + bm1[...]

    @pl.when(pl.program_id(0) == 0)
    def _():
        s_ref[...] = jnp.zeros_like(s_ref)
        ss_ref[...] = jnp.zeros_like(ss_ref)

    s_ref[...] += jnp.sum(m1, axis=0, keepdims=True)
    ss_ref[...] += jnp.sum(m1 * m1, axis=0, keepdims=True)
    hd_ref[...] = jax.nn.sigmoid(_dot(ea, we[...]) + be[...])


def _edge_p2(h2, src, sc2, sh2, w3, b3, wm1x, wm1e, bm1, we=None, be=None):
    ls = h2.shape[1]
    lsn = wm1x.shape[1]
    ins = [_rows(BR_E, ls), _rows(BR_E, 16), _full((1, ls)), _full((1, ls)),
           _full((ls, 19)), _full((1, 19)), _full((16, lsn)),
           _full((19, lsn)), _full((1, lsn))]
    outs = [_rows(BR_E, 19), _full((1, lsn)), _full((1, lsn))]
    oshapes = [_sd((N_E, 19)), _sd((1, lsn)), _sd((1, lsn))]
    args = [h2, src, sc2, sh2, w3, _row(b3), wm1x, wm1e, _row(bm1)]
    if we is None:
        body = _p2_body
    else:
        body = _p2_body_head
        ins += [_full((19, 1)), _full((1, 1))]
        outs.append(_rows(BR_E, 1))
        oshapes.append(_sd((N_E, 1)))
        args += [we, _row(be)]
    return pl.pallas_call(
        body, grid=(N_E // BR_E,), in_specs=ins, out_specs=outs,
        out_shape=oshapes, interpret=_INTERPRET,
    )(*args)


def _p2b_body(src, ea, wm1x, wm1e, bm1, scn, shn, h_ref):
    m1 = _dotb(src[...], wm1x[...]) + _dotb(ea[...], wm1e[...]) + bm1[...]
    h_ref[...] = jnp.maximum(m1 * scn[...] + shn[...], 0.0)


def _edge_p2b(src, ea, wm1x, wm1e, bm1, scn, shn):
    lsn = wm1x.shape[1]
    return pl.pallas_call(
        _p2b_body,
        grid=(N_E // BR_E,),
        in_specs=[_rows(BR_E, 16), _rows(BR_E, 19), _full((16, lsn)),
                  _full((19, lsn)), _full((1, lsn)), _full((1, lsn)),
                  _full((1, lsn))],
        out_specs=_rows(BR_E, lsn),
        out_shape=_sd((N_E, lsn)),
        interpret=_INTERPRET,
    )(src, ea, wm1x, wm1e, _row(bm1), scn, shn)


_S_CB = 15
_S_CPT = _G_NCH // _NS
_S_XTRA = _G_NCH - _S_CPT * _NS
_S_RPT = N_V // _NS


def _make_sc_scatter(ls):
    nsl = ls // 16

    def body(h_hbm, col2d, zero_hbm, out_hbm, idx_v, data_v, accum, sem):
        cid = lax.axis_index("c")
        sid = lax.axis_index("s")

        for sl in range(nsl // _NC):
            c0 = (2 * sl + cid) * 16
            pltpu.sync_copy(zero_hbm.at[pl.ds(0, _S_RPT)],
                            accum.at[pl.ds(sid * _S_RPT, _S_RPT)])
            plsc.subcore_barrier()

            def batch(b, carry):
                ch0 = sid + b * (_S_CB * _NS)
                cps = [pltpu.async_copy(col2d.at[ch0 + j * _NS],
                                        idx_v.at[j], sem)
                       for j in range(_S_CB)]
                for c in cps:
                    c.wait()
                cps = [pltpu.async_copy(
                    h_hbm.at[pl.ds((ch0 + j * _NS) * 128, 128),
                             pl.ds(c0, 16)],
                    data_v.at[pl.ds(j * 128, 128)], sem)
                    for j in range(_S_CB)]
                for c in cps:
                    c.wait()
                cps = [pltpu.async_copy(data_v.at[pl.ds(j * 128, 128)],
                                        accum.at[idx_v.at[j]], sem,
                                        add=True)
                       for j in range(_S_CB)]
                for c in cps:
                    c.wait()
                return carry

            lax.fori_loop(0, _S_CPT // _S_CB, batch, 0)

            @pl.when(sid < _S_XTRA)
            def _():
                ch = sid + _S_CPT * _NS
                pltpu.sync_copy(col2d.at[ch], idx_v.at[0])
                pltpu.async_copy(
                    h_hbm.at[pl.ds(ch * 128, 128), pl.ds(c0, 16)],
                    data_v.at[pl.ds(0, 128)], sem).wait()
                pltpu.sync_copy(data_v.at[pl.ds(0, 128)],
                                accum.at[idx_v.at[0]], add=True)

            plsc.subcore_barrier()
            pltpu.sync_copy(
                accum.at[pl.ds(sid * _S_RPT, _S_RPT)],
                out_hbm.at[pl.ds(sid * _S_RPT, _S_RPT), pl.ds(c0, 16)])
            plsc.subcore_barrier()

    return body


def _sc_scatter(h, col2d, zeros):
    ls = h.shape[1]
    mesh = plsc.VectorSubcoreMesh(core_axis_name="c", subcore_axis_name="s")
    f = pl.kernel(
        _make_sc_scatter(ls), mesh=mesh,
        out_type=_sd((N_V, ls)),
        scratch_types=[pltpu.VMEM((_S_CB, 128), jnp.int32),
                       pltpu.VMEM((_S_CB * 128, 16), f32),
                       pltpu.VMEM_SHARED((N_V, 16), f32),
                       pltpu.SemaphoreType.DMA],
        compiler_params=pltpu.CompilerParams(use_tc_tiling_on_sc=False),
    )
    return f(h, col2d, zeros)


def _sc_counts_body(col2d, ones_hbm, zero_hbm, out_hbm,
                    idx_v, ones_v, accum, sem):
    cid = lax.axis_index("c")
    sid = lax.axis_index("s")

    @pl.when(cid == 0)
    def _():
        pltpu.sync_copy(ones_hbm, ones_v)
        pltpu.sync_copy(zero_hbm.at[pl.ds(0, _S_RPT)],
                        accum.at[pl.ds(sid * _S_RPT, _S_RPT)])
        plsc.subcore_barrier()

        def batch(b, carry):
            ch0 = sid + b * (_S_CB * _NS)
            cps = [pltpu.async_copy(col2d.at[ch0 + j * _NS], idx_v.at[j],
                                    sem)
                   for j in range(_S_CB)]
            for c in cps:
                c.wait()
            cps = [pltpu.async_copy(ones_v, accum.at[idx_v.at[j]], sem,
                                    add=True)
                   for j in range(_S_CB)]
            for c in cps:
                c.wait()
            return carry

        lax.fori_loop(0, _S_CPT // _S_CB, batch, 0)

        @pl.when(sid < _S_XTRA)
        def _():
            ch = sid + _S_CPT * _NS
            pltpu.sync_copy(col2d.at[ch], idx_v.at[0])
            pltpu.sync_copy(ones_v, accum.at[idx_v.at[0]], add=True)

        plsc.subcore_barrier()
        pltpu.sync_copy(accum.at[pl.ds(sid * _S_RPT, _S_RPT)],
                        out_hbm.at[pl.ds(sid * _S_RPT, _S_RPT)])


def _sc_counts(col2d, ones, zeros):
    mesh = plsc.VectorSubcoreMesh(core_axis_name="c", subcore_axis_name="s")
    f = pl.kernel(
        _sc_counts_body, mesh=mesh,
        out_type=_sd((N_V, 16)),
        scratch_types=[pltpu.VMEM((_S_CB, 128), jnp.int32),
                       pltpu.VMEM((128, 16), f32),
                       pltpu.VMEM_SHARED((N_V, 16), f32),
                       pltpu.SemaphoreType.DMA],
        compiler_params=pltpu.CompilerParams(use_tc_tiling_on_sc=False),
    )
    return f(col2d, ones, zeros)


def _p4_body(x, seg, cnt, wm2x, wm2s, bm2, m2_ref, s_ref, ss_ref):
    c = jnp.maximum(cnt[...][:, 0:1], 1.0)
    segm = seg[...] * (1.0 / c)
    m2 = _dot(x[...], wm2x[...]) + _dot(segm, wm2s[...]) + bm2[...]
    m2_ref[...] = m2

    @pl.when(pl.program_id(0) == 0)
    def _():
        s_ref[...] = jnp.zeros_like(s_ref)
        ss_ref[...] = jnp.zeros_like(ss_ref)

    s_ref[...] += jnp.sum(m2, axis=0, keepdims=True)
    ss_ref[...] += jnp.sum(m2 * m2, axis=0, keepdims=True)


def _node_p4(x, seg, cnt16, wm2x, wm2s, bm2):
    ls = seg.shape[1]
    e2 = wm2x.shape[1]
    return pl.pallas_call(
        _p4_body,
        grid=(N_V // BR_N,),
        in_specs=[_rows(BR_N, 16), _rows(BR_N, ls), _rows(BR_N, 16),
                  _full((16, e2)), _full((ls, e2)), _full((1, e2))],
        out_specs=[_rows(BR_N, e2), _full((1, e2)), _full((1, e2))],
        out_shape=[_sd((N_V, e2)), _sd((1, e2)), _sd((1, e2))],
        interpret=_INTERPRET,
    )(x, seg, cnt16, wm2x, wm2s, _row(bm2))


def _p5_body_bsum(m2, oh, sc, sh, wlin, blin, x_ref, bs_ref, bc_ref):
    a = jnp.maximum(m2[...] * sc[...] + sh[...], 0.0)
    xn = jnp.maximum(_dot(a, wlin[...]) + blin[...], 0.0)
    x_ref[...] = xn

    @pl.when(pl.program_id(0) == 0)
    def _():
        bs_ref[...] = jnp.zeros_like(bs_ref)
        bc_ref[...] = jnp.zeros_like(bc_ref)

    o = oh[...]
    bs_ref[...] += lax.dot_general(o, xn, (((0,), (0,)), ((), ())),
                                   preferred_element_type=f32)
    ones = jnp.ones((o.shape[0], 1), f32)
    bc_ref[...] += lax.dot_general(o, ones, (((0,), (0,)), ((), ())),
                                   preferred_element_type=f32)


def _p5_body_head(m2, sc, sh, wlin, blin, wx, bx, x_ref, y_ref):
    a = jnp.maximum(m2[...] * sc[...] + sh[...], 0.0)
    xn = jnp.maximum(_dot(a, wlin[...]) + blin[...], 0.0)
    x_ref[...] = xn
    y_ref[...] = jax.nn.sigmoid(_dot(xn, wx[...]) + bx[...])


def _node_p5(m2, sc, sh, wlin, blin, onehot=None, wx=None, bx=None):
    e2 = m2.shape[1]
    if onehot is not None:
        return pl.pallas_call(
            _p5_body_bsum,
            grid=(N_V // BR_N,),
            in_specs=[_rows(BR_N, e2), _rows(BR_N, NG), _full((1, e2)),
                      _full((1, e2)), _full((e2, 16)), _full((1, 16))],
            out_specs=[_rows(BR_N, 16), _full((NG, 16)), _full((NG, 1))],
            out_shape=[_sd((N_V, 16)), _sd((NG, 16)), _sd((NG, 1))],
            interpret=_INTERPRET,
        )(m2, onehot, sc, sh, wlin, _row(blin))
    return pl.pallas_call(
        _p5_body_head,
        grid=(N_V // BR_N,),
        in_specs=[_rows(BR_N, e2), _full((1, e2)), _full((1, e2)),
                  _full((e2, 16)), _full((1, 16)), _full((16, 1)),
                  _full((1, 1))],
        out_specs=[_rows(BR_N, 16), _rows(BR_N, 1)],
        out_shape=[_sd((N_V, 16)), _sd((N_V, 1))],
        interpret=_INTERPRET,
    )(m2, sc, sh, wlin, _row(blin), wx, _row(bx))


def _p6_body(u, bs, bc, w1u, w1s, b1, g1, bb1, w2, b2, g2, bb2, w3, b3,
             out_ref):
    cnt = jnp.maximum(bc[...], 1.0)
    seg = bs[...] * (1.0 / cnt)
    h = _dot(u[...], w1u[...]) + _dot(seg, w1s[...]) + b1[...]

    def _bn(h, g, bb):
        mu = jnp.mean(h, axis=0, keepdims=True)
        var = jnp.mean(h * h, axis=0, keepdims=True) - mu * mu
        return (h - mu) / jnp.sqrt(var + 1e-5) * g[...] + bb[...]

    a1 = jnp.maximum(_bn(h, g1, bb1), 0.0)
    h2 = _dot(a1, w2[...]) + b2[...]
    a2 = jnp.maximum(_bn(h2, g2, bb2), 0.0)
    out_ref[...] = _dot(a2, w3[...]) + b3[...]


def _glob_p6(u, bs, bc, gp):
    w1u, w1s = gp["l1"]["w"][:7], gp["l1"]["w"][7:]
    return pl.pallas_call(
        _p6_body,
        in_specs=[_full((NG, 7)), _full((NG, 16)), _full((NG, 1)),
                  _full((7, 128)), _full((16, 128)), _full((1, 128)),
                  _full((1, 128)), _full((1, 128)), _full((128, 128)),
                  _full((1, 128)), _full((1, 128)), _full((1, 128)),
                  _full((128, 7)), _full((1, 7))],
        out_specs=_full((NG, 7)),
        out_shape=_sd((NG, 7)),
        interpret=_INTERPRET,
    )(u, bs, bc, w1u, w1s, _row(gp["l1"]["b"]), _row(gp["bn1"]["g"]),
      _row(gp["bn1"]["b"]), gp["l2"]["w"], _row(gp["l2"]["b"]),
      _row(gp["bn2"]["g"]), _row(gp["bn2"]["b"]), gp["l3"]["w"],
      _row(gp["l3"]["b"]))


_G_CPW = _G_NCH // _NW
_G_XTRA = _G_NCH - _G_CPW * _NW
_G_CB = 15
_G_NB = _G_CPW // _G_CB


def _sc_gather_body(x_hbm, row2d, col2d, src_hbm, dst_hbm,
                    idx_v, rows_v, sem):
    wid = lax.axis_index("s") * _NC + lax.axis_index("c")

    def run(ih2d, oh):
        def batch(b, carry):
            c0 = wid + b * (_G_CB * _NW)
            cps = [pltpu.async_copy(ih2d.at[c0 + j * _NW], idx_v.at[j], sem)
                   for j in range(_G_CB)]
            for c in cps:
                c.wait()
            cps = [pltpu.async_copy(x_hbm.at[idx_v.at[j]],
                                    rows_v.at[pl.ds(j * 128, 128)], sem)
                   for j in range(_G_CB)]
            for c in cps:
                c.wait()
            cps = [pltpu.async_copy(rows_v.at[pl.ds(j * 128, 128)],
                                    oh.at[pl.ds((c0 + j * _NW) * 128, 128)],
                                    sem)
                   for j in range(_G_CB)]
            for c in cps:
                c.wait()
            return carry

        lax.fori_loop(0, _G_NB, batch, 0)

        @pl.when(wid < _G_XTRA)
        def _():
            c = wid + _G_CPW * _NW
            pltpu.sync_copy(ih2d.at[c], idx_v.at[0])
            pltpu.async_copy(x_hbm.at[idx_v.at[0]],
                             rows_v.at[pl.ds(0, 128)], sem).wait()
            pltpu.sync_copy(rows_v.at[pl.ds(0, 128)],
                            oh.at[pl.ds(c * 128, 128)])

    run(row2d, src_hbm)
    run(col2d, dst_hbm)


def _sc_gather(x, row2d, col2d):
    mesh = plsc.VectorSubcoreMesh(core_axis_name="c", subcore_axis_name="s")
    f = pl.kernel(
        _sc_gather_body, mesh=mesh,
        out_type=[_sd((N_E, 16)), _sd((N_E, 16))],
        scratch_types=[pltpu.VMEM((_G_CB, 128), jnp.int32),
                       pltpu.VMEM((_G_CB * 128, 16), f32),
                       pltpu.SemaphoreType.DMA],
        compiler_params=pltpu.CompilerParams(use_tc_tiling_on_sc=False),
    )
    return f(x, row2d, col2d)


def kernel(x, edge_index, edge_attr, u, batch, params):
    row, col = edge_index[0], edge_index[1]
    row2d = row.reshape(N_E // 128, 128)
    col2d = col.reshape(N_E // 128, 128)
    onehot = (batch[:, None] == jnp.arange(NG, dtype=jnp.int32)[None, :]
              ).astype(f32)
    zeros = jnp.zeros((_S_RPT, 16), f32)
    ones = jnp.ones((128, 16), f32)
    cnt16 = _sc_counts(col2d, ones, zeros)
    ea = edge_attr
    y_pred = None
    head = None
    for li, name in enumerate(("ml1", "ml2", "ml3")):
        p = params[name]
        ep, np_, gp = p["edge"], p["node"], p["glob"]
        last = li == 2
        src, dst = _sc_gather(x, row2d, col2d)
        w1 = ep["l1"]["w"].astype(bf16)
        w1s, w1d, w1e = w1[:16], w1[16:32], w1[32:]
        s1, ss1 = _edge_stats1(src, dst, ea, w1s, w1d, w1e, ep["l1"]["b"])
        sc1, sh1 = _fin_stats(s1[0], ss1[0], N_E, ep["bn1"]["g"],
                              ep["bn1"]["b"])
        h2, s2, ss2 = _edge_p1(src, dst, ea, w1s, w1d, w1e, ep["l1"]["b"],
                               sc1, sh1, ep["l2"]["w"].astype(bf16),
                               ep["l2"]["b"])
        sc2, sh2 = _fin_stats(s2[0], ss2[0], N_E, ep["bn2"]["g"],
                              ep["bn2"]["b"])
        wm1 = np_["m1"]["w"].astype(bf16)
        wm1x, wm1e = wm1[:16], wm1[16:]
        if last:
            ea, sn, ssn, head = _edge_p2(
                h2, src, sc2, sh2, ep["lin"]["w"].astype(bf16),
                ep["lin"]["b"],
                wm1x, wm1e, np_["m1"]["b"],
                params["e_lin"]["w"], params["e_lin"]["b"])
        else:
            ea, sn, ssn = _edge_p2(
                h2, src, sc2, sh2, ep["lin"]["w"].astype(bf16),
                ep["lin"]["b"],
                wm1x, wm1e, np_["m1"]["b"])
        scn, shn = _fin_stats(sn[0], ssn[0], N_E, np_["bn1"]["g"],
                              np_["bn1"]["b"])
        h = _edge_p2b(src, ea, wm1x, wm1e, np_["m1"]["b"], scn, shn)
        seg = _sc_scatter(h, col2d, zeros)
        wm2 = np_["m2"]["w"]
        m2, sm, ssm = _node_p4(x, seg, cnt16, wm2[:16], wm2[16:],
                               np_["m2"]["b"])
        scm, shm = _fin_stats(sm[0], ssm[0], N_V, np_["bn2"]["g"],
                              np_["bn2"]["b"])
        if last:
            x, y_pred = _node_p5(m2, scm, shm, np_["lin"]["w"],
                                 np_["lin"]["b"], wx=params["x_lin"]["w"],
                                 bx=params["x_lin"]["b"])
        else:
            x, bs, bc = _node_p5(m2, scm, shm, np_["lin"]["w"],
                                 np_["lin"]["b"], onehot=onehot)
            u = _glob_p6(u, bs, bc, gp)
    return (y_pred, head)


# speedup vs baseline: 2.3586x; 1.0578x over previous
import functools

import jax
import jax.numpy as jnp
from jax import lax
from jax.experimental import pallas as pl
from jax.experimental.pallas import tpu as pltpu
from jax.experimental.pallas import tpu_sc as plsc

_INTERPRET = False

_NC = 2
_NS = 16
_NW = _NC * _NS
_G_NCH = 800000 // 128

N_E = 800000
N_V = 50000
NG = 64
BR_E = 4000
BR_N = 2000
f32 = jnp.float32
bf16 = jnp.bfloat16


def _row(v):
    return v.reshape(1, -1)


def _fin_stats(s, ss, n, g, bbn):
    mu = s / n
    var = ss / n - mu * mu
    isd = 1.0 / jnp.sqrt(var + 1e-5)
    sc = g * isd
    sh = bbn - mu * sc
    return _row(sc), _row(sh)


def _dot(a, b):
    return jnp.dot(a, b, preferred_element_type=f32)


def _dotb(a, b):
    return jnp.dot(a.astype(bf16), b, preferred_element_type=f32)


def _full(shape):
    return pl.BlockSpec(shape, lambda *a: tuple(0 for _ in shape))


def _rows(br, w):
    return pl.BlockSpec((br, w), lambda i: (i, 0))


def _sd(shape):
    return jax.ShapeDtypeStruct(shape, f32)


def _p0_body(src, dst, ea, w1s, w1d, w1e, b1, s_ref, ss_ref):
    h = (_dotb(src[...], w1s[...]) + _dotb(dst[...], w1d[...])
         + _dotb(ea[...], w1e[...]) + b1[...])

    @pl.when(pl.program_id(0) == 0)
    def _():
        s_ref[...] = jnp.zeros_like(s_ref)
        ss_ref[...] = jnp.zeros_like(ss_ref)

    s_ref[...] += jnp.sum(h, axis=0, keepdims=True)
    ss_ref[...] += jnp.sum(h * h, axis=0, keepdims=True)


def _edge_stats1(src, dst, ea, w1s, w1d, w1e, b1):
    ls = w1s.shape[1]
    s, ss = pl.pallas_call(
        _p0_body,
        grid=(N_E // BR_E,),
        in_specs=[_rows(BR_E, 16), _rows(BR_E, 16), _rows(BR_E, 19),
                  _full((16, ls)), _full((16, ls)), _full((19, ls)),
                  _full((1, ls))],
        out_specs=[_full((1, ls)), _full((1, ls))],
        out_shape=[_sd((1, ls))] * 2,
        interpret=_INTERPRET,
    )(src, dst, ea, w1s, w1d, w1e, _row(b1))
    return s, ss


def _p1_body(src, dst, ea, w1s, w1d, w1e, b1, sc1, sh1, w2, b2,
             h2_ref, s_ref, ss_ref):
    h1 = (_dotb(src[...], w1s[...]) + _dotb(dst[...], w1d[...])
          + _dotb(ea[...], w1e[...]) + b1[...])
    a1 = jnp.maximum(h1 * sc1[...] + sh1[...], 0.0)
    h2 = _dotb(a1, w2[...]) + b2[...]
    h2_ref[...] = h2.astype(bf16)

    @pl.when(pl.program_id(0) == 0)
    def _():
        s_ref[...] = jnp.zeros_like(s_ref)
        ss_ref[...] = jnp.zeros_like(ss_ref)

    s_ref[...] += jnp.sum(h2, axis=0, keepdims=True)
    ss_ref[...] += jnp.sum(h2 * h2, axis=0, keepdims=True)


def _edge_p1(src, dst, ea, w1s, w1d, w1e, b1, sc1, sh1, w2, b2):
    ls = w2.shape[1]
    return pl.pallas_call(
        _p1_body,
        grid=(N_E // BR_E,),
        in_specs=[_rows(BR_E, 16), _rows(BR_E, 16), _rows(BR_E, 19),
                  _full((16, ls)), _full((16, ls)), _full((19, ls)),
                  _full((1, ls)), _full((1, ls)), _full((1, ls)),
                  _full((ls, ls)), _full((1, ls))],
        out_specs=[_rows(BR_E, ls), _full((1, ls)), _full((1, ls))],
        out_shape=[jax.ShapeDtypeStruct((N_E, ls), bf16), _sd((1, ls)),
                   _sd((1, ls))],
        interpret=_INTERPRET,
    )(src, dst, ea, w1s, w1d, w1e, _row(b1), sc1, sh1, w2, _row(b2))


def _p2_body(h2, src, sc2, sh2, w3, b3, wm1x, wm1e, bm1,
             ea_ref, s_ref, ss_ref):
    a2 = jnp.maximum(h2[...].astype(f32) * sc2[...] + sh2[...], 0.0)
    ea = jnp.maximum(_dotb(a2, w3[...]) + b3[...], 0.0)
    ea_ref[...] = ea
    m1 = _dotb(src[...], wm1x[...]) + _dotb(ea, wm1e[...]) + bm1[...]

    @pl.when(pl.program_id(0) == 0)
    def _():
        s_ref[...] = jnp.zeros_like(s_ref)
        ss_ref[...] = jnp.zeros_like(ss_ref)

    s_ref[...] += jnp.sum(m1, axis=0, keepdims=True)
    ss_ref[...] += jnp.sum(m1 * m1, axis=0, keepdims=True)


def _p2_body_head(h2, src, sc2, sh2, w3, b3, wm1x, wm1e, bm1, we, be,
                  ea_ref, s_ref, ss_ref, hd_ref):
    a2 = jnp.maximum(h2[...].astype(f32) * sc2[...] + sh2[...], 0.0)
    ea = jnp.maximum(_dotb(a2, w3[...]) + b3[...], 0.0)
    ea_ref[...] = ea
    m1 = _dotb(src[...], wm1x[...]) + _dotb(ea, wm1e[...]) + bm1[...]

    @pl.when(pl.program_id(0) == 0)
    def _():
        s_ref[...] = jnp.zeros_like(s_ref)
        ss_ref[...] = jnp.zeros_like(ss_ref)

    s_ref[...] += jnp.sum(m1, axis=0, keepdims=True)
    ss_ref[...] += jnp.sum(m1 * m1, axis=0, keepdims=True)
    hd_ref[...] = jax.nn.sigmoid(_dot(ea, we[...]) + be[...])


def _edge_p2(h2, src, sc2, sh2, w3, b3, wm1x, wm1e, bm1, we=None, be=None):
    ls = h2.shape[1]
    lsn = wm1x.shape[1]
    ins = [_rows(BR_E, ls), _rows(BR_E, 16), _full((1, ls)), _full((1, ls)),
           _full((ls, 19)), _full((1, 19)), _full((16, lsn)),
           _full((19, lsn)), _full((1, lsn))]
    outs = [_rows(BR_E, 19), _full((1, lsn)), _full((1, lsn))]
    oshapes = [_sd((N_E, 19)), _sd((1, lsn)), _sd((1, lsn))]
    args = [h2, src, sc2, sh2, w3, _row(b3), wm1x, wm1e, _row(bm1)]
    if we is None:
        body = _p2_body
    else:
        body = _p2_body_head
        ins += [_full((19, 1)), _full((1, 1))]
        outs.append(_rows(BR_E, 1))
        oshapes.append(_sd((N_E, 1)))
        args += [we, _row(be)]
    return pl.pallas_call(
        body, grid=(N_E // BR_E,), in_specs=ins, out_specs=outs,
        out_shape=oshapes, interpret=_INTERPRET,
    )(*args)


def _p2b_body(src, ea, wm1x, wm1e, bm1, scn, shn, *h_refs):
    m1 = _dotb(src[...], wm1x[...]) + _dotb(ea[...], wm1e[...]) + bm1[...]
    h = jnp.maximum(m1 * scn[...] + shn[...], 0.0)
    w = 128 if len(h_refs) > 1 else h.shape[1]
    for i, r in enumerate(h_refs):
        r[...] = h[:, i * w:(i + 1) * w]


def _edge_p2b(src, ea, wm1x, wm1e, bm1, scn, shn):
    lsn = wm1x.shape[1]
    npart = max(1, lsn // 128)
    pw = lsn // npart
    return pl.pallas_call(
        _p2b_body,
        grid=(N_E // BR_E,),
        in_specs=[_rows(BR_E, 16), _rows(BR_E, 19), _full((16, lsn)),
                  _full((19, lsn)), _full((1, lsn)), _full((1, lsn)),
                  _full((1, lsn))],
        out_specs=[_rows(BR_E, pw)] * npart,
        out_shape=[_sd((N_E, pw))] * npart,
        interpret=_INTERPRET,
    )(src, ea, wm1x, wm1e, _row(bm1), scn, shn)


_S_CB = 15
_S_CPT = _G_NCH // _NS
_S_XTRA = _G_NCH - _S_CPT * _NS
_S_RPT = N_V // _NS


def _make_sc_scatter(ls, npart):
    nsl = ls // 16
    spp = nsl // npart

    def body(*refs):
        h_parts = refs[:npart]
        col2d, zero_hbm = refs[npart], refs[npart + 1]
        out_parts = refs[npart + 2:2 * npart + 2]
        idx_v, data_v, accum, sem = refs[2 * npart + 2:]
        cid = lax.axis_index("c")
        sid = lax.axis_index("s")

        for part in range(npart):
            h_hbm = h_parts[part]
            out_hbm = out_parts[part]

            def slice_body(k, carry):
                si = part * spp + k
                c0 = k * 16
                own = si % _NC == cid

                @pl.when(own)
                def _():
                    pltpu.sync_copy(zero_hbm.at[pl.ds(0, _S_RPT)],
                                    accum.at[pl.ds(sid * _S_RPT, _S_RPT)])

                plsc.subcore_barrier()

                def batch(b, carry2):
                    ch0 = sid + b * (_S_CB * _NS)
                    cps = [pltpu.async_copy(col2d.at[ch0 + j * _NS],
                                            idx_v.at[j], sem)
                           for j in range(_S_CB)]
                    for c in cps:
                        c.wait()
                    cps = [pltpu.async_copy(
                        h_hbm.at[pl.ds((ch0 + j * _NS) * 128, 128),
                                 pl.ds(c0, 16)],
                        data_v.at[pl.ds(j * 128, 128)], sem)
                        for j in range(_S_CB)]
                    for c in cps:
                        c.wait()
                    cps = [pltpu.async_copy(data_v.at[pl.ds(j * 128, 128)],
                                            accum.at[idx_v.at[j]], sem,
                                            add=True)
                           for j in range(_S_CB)]
                    for c in cps:
                        c.wait()
                    return carry2

                @pl.when(own)
                def _():
                    lax.fori_loop(0, _S_CPT // _S_CB, batch, 0)

                    @pl.when(sid < _S_XTRA)
                    def _():
                        ch = sid + _S_CPT * _NS
                        pltpu.sync_copy(col2d.at[ch], idx_v.at[0])
                        pltpu.async_copy(
                            h_hbm.at[pl.ds(ch * 128, 128), pl.ds(c0, 16)],
                            data_v.at[pl.ds(0, 128)], sem).wait()
                        pltpu.sync_copy(data_v.at[pl.ds(0, 128)],
                                        accum.at[idx_v.at[0]], add=True)

                plsc.subcore_barrier()

                @pl.when(own)
                def _():
                    pltpu.sync_copy(
                        accum.at[pl.ds(sid * _S_RPT, _S_RPT)],
                        out_hbm.at[pl.ds(sid * _S_RPT, _S_RPT),
                                   pl.ds(c0, 16)])

                plsc.subcore_barrier()
                return carry

            lax.fori_loop(0, spp, slice_body, 0)

    return body


def _sc_scatter(h_parts, col2d, zeros):
    npart = len(h_parts)
    pw = h_parts[0].shape[1]
    ls = npart * pw
    mesh = plsc.VectorSubcoreMesh(core_axis_name="c", subcore_axis_name="s")
    f = pl.kernel(
        _make_sc_scatter(ls, npart), mesh=mesh,
        out_type=[_sd((N_V, pw))] * npart,
        scratch_types=[pltpu.VMEM((_S_CB, 128), jnp.int32),
                       pltpu.VMEM((_S_CB * 128, 16), f32),
                       pltpu.VMEM_SHARED((N_V, 16), f32),
                       pltpu.SemaphoreType.DMA],
        compiler_params=pltpu.CompilerParams(use_tc_tiling_on_sc=False),
    )
    out = f(*h_parts, col2d, zeros)
    return out if isinstance(out, (list, tuple)) else [out]


def _sc_counts_body(col2d, ones_hbm, zero_hbm, out_hbm,
                    idx_v, ones_v, accum, sem):
    cid = lax.axis_index("c")
    sid = lax.axis_index("s")

    @pl.when(cid == 0)
    def _():
        pltpu.sync_copy(ones_hbm, ones_v)
        pltpu.sync_copy(zero_hbm.at[pl.ds(0, _S_RPT)],
                        accum.at[pl.ds(sid * _S_RPT, _S_RPT)])
        plsc.subcore_barrier()

        def batch(b, carry):
            ch0 = sid + b * (_S_CB * _NS)
            cps = [pltpu.async_copy(col2d.at[ch0 + j * _NS], idx_v.at[j],
                                    sem)
                   for j in range(_S_CB)]
            for c in cps:
                c.wait()
            cps = [pltpu.async_copy(ones_v, accum.at[idx_v.at[j]], sem,
                                    add=True)
                   for j in range(_S_CB)]
            for c in cps:
                c.wait()
            return carry

        lax.fori_loop(0, _S_CPT // _S_CB, batch, 0)

        @pl.when(sid < _S_XTRA)
        def _():
            ch = sid + _S_CPT * _NS
            pltpu.sync_copy(col2d.at[ch], idx_v.at[0])
            pltpu.sync_copy(ones_v, accum.at[idx_v.at[0]], add=True)

        plsc.subcore_barrier()
        pltpu.sync_copy(accum.at[pl.ds(sid * _S_RPT, _S_RPT)],
                        out_hbm.at[pl.ds(sid * _S_RPT, _S_RPT)])


def _sc_counts(col2d, ones, zeros):
    mesh = plsc.VectorSubcoreMesh(core_axis_name="c", subcore_axis_name="s")
    f = pl.kernel(
        _sc_counts_body, mesh=mesh,
        out_type=_sd((N_V, 16)),
        scratch_types=[pltpu.VMEM((_S_CB, 128), jnp.int32),
                       pltpu.VMEM((128, 16), f32),
                       pltpu.VMEM_SHARED((N_V, 16), f32),
                       pltpu.SemaphoreType.DMA],
        compiler_params=pltpu.CompilerParams(use_tc_tiling_on_sc=False),
    )
    return f(col2d, ones, zeros)


def _p4_body(x, cnt, wm2x, bm2, *rest):
    nseg = (len(rest) - 3) // 2
    segs = rest[:nseg]
    wm2ss = rest[nseg:2 * nseg]
    m2_ref, s_ref, ss_ref = rest[2 * nseg:]
    c = jnp.maximum(cnt[...][:, 0:1], 1.0)
    inv = 1.0 / c
    m2 = _dot(x[...], wm2x[...]) + bm2[...]
    for sg, w in zip(segs, wm2ss):
        m2 = m2 + _dot(sg[...] * inv, w[...])
    m2_ref[...] = m2

    @pl.when(pl.program_id(0) == 0)
    def _():
        s_ref[...] = jnp.zeros_like(s_ref)
        ss_ref[...] = jnp.zeros_like(ss_ref)

    s_ref[...] += jnp.sum(m2, axis=0, keepdims=True)
    ss_ref[...] += jnp.sum(m2 * m2, axis=0, keepdims=True)


def _node_p4(x, seg_parts, cnt16, wm2x, wm2s, bm2):
    pw = seg_parts[0].shape[1]
    e2 = wm2x.shape[1]
    npart = len(seg_parts)
    wparts = [wm2s[i * pw:(i + 1) * pw] for i in range(npart)]
    return pl.pallas_call(
        _p4_body,
        grid=(N_V // BR_N,),
        in_specs=([_rows(BR_N, 16), _rows(BR_N, 16), _full((16, e2)),
                   _full((1, e2))]
                  + [_rows(BR_N, pw)] * npart
                  + [_full((pw, e2))] * npart),
        out_specs=[_rows(BR_N, e2), _full((1, e2)), _full((1, e2))],
        out_shape=[_sd((N_V, e2)), _sd((1, e2)), _sd((1, e2))],
        interpret=_INTERPRET,
    )(x, cnt16, wm2x, _row(bm2), *seg_parts, *wparts)


def _p5_body_bsum(m2, oh, sc, sh, wlin, blin, x_ref, bs_ref, bc_ref):
    a = jnp.maximum(m2[...] * sc[...] + sh[...], 0.0)
    xn = jnp.maximum(_dot(a, wlin[...]) + blin[...], 0.0)
    x_ref[...] = xn

    @pl.when(pl.program_id(0) == 0)
    def _():
        bs_ref[...] = jnp.zeros_like(bs_ref)
        bc_ref[...] = jnp.zeros_like(bc_ref)

    o = oh[...]
    bs_ref[...] += lax.dot_general(o, xn, (((0,), (0,)), ((), ())),
                                   preferred_element_type=f32)
    ones = jnp.ones((o.shape[0], 1), f32)
    bc_ref[...] += lax.dot_general(o, ones, (((0,), (0,)), ((), ())),
                                   preferred_element_type=f32)


def _p5_body_head(m2, sc, sh, wlin, blin, wx, bx, x_ref, y_ref):
    a = jnp.maximum(m2[...] * sc[...] + sh[...], 0.0)
    xn = jnp.maximum(_dot(a, wlin[...]) + blin[...], 0.0)
    x_ref[...] = xn
    y_ref[...] = jax.nn.sigmoid(_dot(xn, wx[...]) + bx[...])


def _node_p5(m2, sc, sh, wlin, blin, onehot=None, wx=None, bx=None):
    e2 = m2.shape[1]
    if onehot is not None:
        return pl.pallas_call(
            _p5_body_bsum,
            grid=(N_V // BR_N,),
            in_specs=[_rows(BR_N, e2), _rows(BR_N, NG), _full((1, e2)),
                      _full((1, e2)), _full((e2, 16)), _full((1, 16))],
            out_specs=[_rows(BR_N, 16), _full((NG, 16)), _full((NG, 1))],
            out_shape=[_sd((N_V, 16)), _sd((NG, 16)), _sd((NG, 1))],
            interpret=_INTERPRET,
        )(m2, onehot, sc, sh, wlin, _row(blin))
    return pl.pallas_call(
        _p5_body_head,
        grid=(N_V // BR_N,),
        in_specs=[_rows(BR_N, e2), _full((1, e2)), _full((1, e2)),
                  _full((e2, 16)), _full((1, 16)), _full((16, 1)),
                  _full((1, 1))],
        out_specs=[_rows(BR_N, 16), _rows(BR_N, 1)],
        out_shape=[_sd((N_V, 16)), _sd((N_V, 1))],
        interpret=_INTERPRET,
    )(m2, sc, sh, wlin, _row(blin), wx, _row(bx))


def _p6_body(u, bs, bc, w1u, w1s, b1, g1, bb1, w2, b2, g2, bb2, w3, b3,
             out_ref):
    cnt = jnp.maximum(bc[...], 1.0)
    seg = bs[...] * (1.0 / cnt)
    h = _dot(u[...], w1u[...]) + _dot(seg, w1s[...]) + b1[...]

    def _bn(h, g, bb):
        mu = jnp.mean(h, axis=0, keepdims=True)
        var = jnp.mean(h * h, axis=0, keepdims=True) - mu * mu
        return (h - mu) / jnp.sqrt(var + 1e-5) * g[...] + bb[...]

    a1 = jnp.maximum(_bn(h, g1, bb1), 0.0)
    h2 = _dot(a1, w2[...]) + b2[...]
    a2 = jnp.maximum(_bn(h2, g2, bb2), 0.0)
    out_ref[...] = _dot(a2, w3[...]) + b3[...]


def _glob_p6(u, bs, bc, gp):
    w1u, w1s = gp["l1"]["w"][:7], gp["l1"]["w"][7:]
    return pl.pallas_call(
        _p6_body,
        in_specs=[_full((NG, 7)), _full((NG, 16)), _full((NG, 1)),
                  _full((7, 128)), _full((16, 128)), _full((1, 128)),
                  _full((1, 128)), _full((1, 128)), _full((128, 128)),
                  _full((1, 128)), _full((1, 128)), _full((1, 128)),
                  _full((128, 7)), _full((1, 7))],
        out_specs=_full((NG, 7)),
        out_shape=_sd((NG, 7)),
        interpret=_INTERPRET,
    )(u, bs, bc, w1u, w1s, _row(gp["l1"]["b"]), _row(gp["bn1"]["g"]),
      _row(gp["bn1"]["b"]), gp["l2"]["w"], _row(gp["l2"]["b"]),
      _row(gp["bn2"]["g"]), _row(gp["bn2"]["b"]), gp["l3"]["w"],
      _row(gp["l3"]["b"]))


_G_CPW = _G_NCH // _NW
_G_XTRA = _G_NCH - _G_CPW * _NW
_G_CB = 15
_G_NB = _G_CPW // _G_CB


def _sc_gather_body(x_hbm, row2d, col2d, src_hbm, dst_hbm,
                    idx_v, rows_v, sem):
    wid = lax.axis_index("s") * _NC + lax.axis_index("c")

    def run(ih2d, oh):
        def batch(b, carry):
            c0 = wid + b * (_G_CB * _NW)
            cps = [pltpu.async_copy(ih2d.at[c0 + j * _NW], idx_v.at[j], sem)
                   for j in range(_G_CB)]
            for c in cps:
                c.wait()
            cps = [pltpu.async_copy(x_hbm.at[idx_v.at[j]],
                                    rows_v.at[pl.ds(j * 128, 128)], sem)
                   for j in range(_G_CB)]
            for c in cps:
                c.wait()
            cps = [pltpu.async_copy(rows_v.at[pl.ds(j * 128, 128)],
                                    oh.at[pl.ds((c0 + j * _NW) * 128, 128)],
                                    sem)
                   for j in range(_G_CB)]
            for c in cps:
                c.wait()
            return carry

        lax.fori_loop(0, _G_NB, batch, 0)

        @pl.when(wid < _G_XTRA)
        def _():
            c = wid + _G_CPW * _NW
            pltpu.sync_copy(ih2d.at[c], idx_v.at[0])
            pltpu.async_copy(x_hbm.at[idx_v.at[0]],
                             rows_v.at[pl.ds(0, 128)], sem).wait()
            pltpu.sync_copy(rows_v.at[pl.ds(0, 128)],
                            oh.at[pl.ds(c * 128, 128)])

    run(row2d, src_hbm)
    run(col2d, dst_hbm)


def _sc_gather(x, row2d, col2d):
    mesh = plsc.VectorSubcoreMesh(core_axis_name="c", subcore_axis_name="s")
    f = pl.kernel(
        _sc_gather_body, mesh=mesh,
        out_type=[_sd((N_E, 16)), _sd((N_E, 16))],
        scratch_types=[pltpu.VMEM((_G_CB, 128), jnp.int32),
                       pltpu.VMEM((_G_CB * 128, 16), f32),
                       pltpu.SemaphoreType.DMA],
        compiler_params=pltpu.CompilerParams(use_tc_tiling_on_sc=False),
    )
    return f(x, row2d, col2d)


def kernel(x, edge_index, edge_attr, u, batch, params):
    row, col = edge_index[0], edge_index[1]
    row2d = row.reshape(N_E // 128, 128)
    col2d = col.reshape(N_E // 128, 128)
    onehot = (batch[:, None] == jnp.arange(NG, dtype=jnp.int32)[None, :]
              ).astype(f32)
    zeros = jnp.zeros((_S_RPT, 16), f32)
    ones = jnp.ones((128, 16), f32)
    cnt16 = _sc_counts(col2d, ones, zeros)
    ea = edge_attr
    y_pred = None
    head = None
    for li, name in enumerate(("ml1", "ml2", "ml3")):
        p = params[name]
        ep, np_, gp = p["edge"], p["node"], p["glob"]
        last = li == 2
        src, dst = _sc_gather(x, row2d, col2d)
        w1 = ep["l1"]["w"].astype(bf16)
        w1s, w1d, w1e = w1[:16], w1[16:32], w1[32:]
        s1, ss1 = _edge_stats1(src, dst, ea, w1s, w1d, w1e, ep["l1"]["b"])
        sc1, sh1 = _fin_stats(s1[0], ss1[0], N_E, ep["bn1"]["g"],
                              ep["bn1"]["b"])
        h2, s2, ss2 = _edge_p1(src, dst, ea, w1s, w1d, w1e, ep["l1"]["b"],
                               sc1, sh1, ep["l2"]["w"].astype(bf16),
                               ep["l2"]["b"])
        sc2, sh2 = _fin_stats(s2[0], ss2[0], N_E, ep["bn2"]["g"],
                              ep["bn2"]["b"])
        wm1 = np_["m1"]["w"].astype(bf16)
        wm1x, wm1e = wm1[:16], wm1[16:]
        if last:
            ea, sn, ssn, head = _edge_p2(
                h2, src, sc2, sh2, ep["lin"]["w"].astype(bf16),
                ep["lin"]["b"],
                wm1x, wm1e, np_["m1"]["b"],
                params["e_lin"]["w"], params["e_lin"]["b"])
        else:
            ea, sn, ssn = _edge_p2(
                h2, src, sc2, sh2, ep["lin"]["w"].astype(bf16),
                ep["lin"]["b"],
                wm1x, wm1e, np_["m1"]["b"])
        scn, shn = _fin_stats(sn[0], ssn[0], N_E, np_["bn1"]["g"],
                              np_["bn1"]["b"])
        h_parts = _edge_p2b(src, ea, wm1x, wm1e, np_["m1"]["b"], scn, shn)
        if not isinstance(h_parts, (list, tuple)):
            h_parts = [h_parts]
        seg_parts = _sc_scatter(h_parts, col2d, zeros)
        wm2 = np_["m2"]["w"]
        m2, sm, ssm = _node_p4(x, seg_parts, cnt16, wm2[:16], wm2[16:],
                               np_["m2"]["b"])
        scm, shm = _fin_stats(sm[0], ssm[0], N_V, np_["bn2"]["g"],
                              np_["bn2"]["b"])
        if last:
            x, y_pred = _node_p5(m2, scm, shm, np_["lin"]["w"],
                                 np_["lin"]["b"], wx=params["x_lin"]["w"],
                                 bx=params["x_lin"]["b"])
        else:
            x, bs, bc = _node_p5(m2, scm, shm, np_["lin"]["w"],
                                 np_["lin"]["b"], onehot=onehot)
            u = _glob_p6(u, bs, bc, gp)
    return (y_pred, head)


# speedup vs baseline: 2.3898x; 1.0132x over previous
import functools

import jax
import jax.numpy as jnp
from jax import lax
from jax.experimental import pallas as pl
from jax.experimental.pallas import tpu as pltpu
from jax.experimental.pallas import tpu_sc as plsc

_INTERPRET = False

_NC = 2
_NS = 16
_NW = _NC * _NS
_G_NCH = 800000 // 128

N_E = 800000
N_V = 50000
NG = 64
BR_E = 4000
BR_N = 2000
f32 = jnp.float32
bf16 = jnp.bfloat16


def _row(v):
    return v.reshape(1, -1)


def _fin_stats(s, ss, n, g, bbn):
    mu = s / n
    var = ss / n - mu * mu
    isd = 1.0 / jnp.sqrt(var + 1e-5)
    sc = g * isd
    sh = bbn - mu * sc
    return _row(sc), _row(sh)


def _dot(a, b):
    return jnp.dot(a, b, preferred_element_type=f32)


def _dotb(a, b):
    return jnp.dot(a.astype(bf16), b, preferred_element_type=f32)


def _full(shape):
    return pl.BlockSpec(shape, lambda *a: tuple(0 for _ in shape))


def _rows(br, w):
    return pl.BlockSpec((br, w), lambda i: (i, 0))


def _sd(shape):
    return jax.ShapeDtypeStruct(shape, f32)


def _p0_body(src, dst, ea, w1s, w1d, w1e, b1, s_ref, ss_ref):
    h = (_dotb(src[...][:, :16], w1s[...]) + _dotb(dst[...][:, :16],
         w1d[...]) + _dotb(ea[...], w1e[...]) + b1[...])

    @pl.when(pl.program_id(0) == 0)
    def _():
        s_ref[...] = jnp.zeros_like(s_ref)
        ss_ref[...] = jnp.zeros_like(ss_ref)

    s_ref[...] += jnp.sum(h, axis=0, keepdims=True)
    ss_ref[...] += jnp.sum(h * h, axis=0, keepdims=True)


def _edge_stats1(src, dst, ea, w1s, w1d, w1e, b1):
    ls = w1s.shape[1]
    s, ss = pl.pallas_call(
        _p0_body,
        grid=(N_E // BR_E,),
        in_specs=[_rows(BR_E, 32), _rows(BR_E, 32), _rows(BR_E, 19),
                  _full((16, ls)), _full((16, ls)), _full((19, ls)),
                  _full((1, ls))],
        out_specs=[_full((1, ls)), _full((1, ls))],
        out_shape=[_sd((1, ls))] * 2,
        interpret=_INTERPRET,
    )(src, dst, ea, w1s, w1d, w1e, _row(b1))
    return s, ss


def _p1_body(src, dst, ea, w1s, w1d, w1e, b1, sc1, sh1, w2, b2,
             h2_ref, s_ref, ss_ref):
    h1 = (_dotb(src[...][:, :16], w1s[...]) + _dotb(dst[...][:, :16],
          w1d[...]) + _dotb(ea[...], w1e[...]) + b1[...])
    a1 = jnp.maximum(h1 * sc1[...] + sh1[...], 0.0)
    h2 = _dotb(a1, w2[...]) + b2[...]
    h2_ref[...] = h2.astype(bf16)

    @pl.when(pl.program_id(0) == 0)
    def _():
        s_ref[...] = jnp.zeros_like(s_ref)
        ss_ref[...] = jnp.zeros_like(ss_ref)

    s_ref[...] += jnp.sum(h2, axis=0, keepdims=True)
    ss_ref[...] += jnp.sum(h2 * h2, axis=0, keepdims=True)


def _edge_p1(src, dst, ea, w1s, w1d, w1e, b1, sc1, sh1, w2, b2):
    ls = w2.shape[1]
    return pl.pallas_call(
        _p1_body,
        grid=(N_E // BR_E,),
        in_specs=[_rows(BR_E, 32), _rows(BR_E, 32), _rows(BR_E, 19),
                  _full((16, ls)), _full((16, ls)), _full((19, ls)),
                  _full((1, ls)), _full((1, ls)), _full((1, ls)),
                  _full((ls, ls)), _full((1, ls))],
        out_specs=[_rows(BR_E, ls), _full((1, ls)), _full((1, ls))],
        out_shape=[jax.ShapeDtypeStruct((N_E, ls), bf16), _sd((1, ls)),
                   _sd((1, ls))],
        interpret=_INTERPRET,
    )(src, dst, ea, w1s, w1d, w1e, _row(b1), sc1, sh1, w2, _row(b2))


def _p2_body(h2, src, sc2, sh2, w3, b3, wm1x, wm1e, bm1,
             ea_ref, s_ref, ss_ref):
    a2 = jnp.maximum(h2[...].astype(f32) * sc2[...] + sh2[...], 0.0)
    ea = jnp.maximum(_dotb(a2, w3[...]) + b3[...], 0.0)
    ea_ref[...] = ea.astype(bf16)
    m1 = (_dotb(src[...][:, :16], wm1x[...]) + _dotb(ea, wm1e[...])
          + bm1[...])

    @pl.when(pl.program_id(0) == 0)
    def _():
        s_ref[...] = jnp.zeros_like(s_ref)
        ss_ref[...] = jnp.zeros_like(ss_ref)

    s_ref[...] += jnp.sum(m1, axis=0, keepdims=True)
    ss_ref[...] += jnp.sum(m1 * m1, axis=0, keepdims=True)


def _p2_body_head(h2, src, sc2, sh2, w3, b3, wm1x, wm1e, bm1, we, be,
                  ea_ref, s_ref, ss_ref, hd_ref):
    a2 = jnp.maximum(h2[...].astype(f32) * sc2[...] + sh2[...], 0.0)
    ea = jnp.maximum(_dotb(a2, w3[...]) + b3[...], 0.0)
    ea_ref[...] = ea.astype(bf16)
    m1 = (_dotb(src[...][:, :16], wm1x[...]) + _dotb(ea, wm1e[...])
          + bm1[...])

    @pl.when(pl.program_id(0) == 0)
    def _():
        s_ref[...] = jnp.zeros_like(s_ref)
        ss_ref[...] = jnp.zeros_like(ss_ref)

    s_ref[...] += jnp.sum(m1, axis=0, keepdims=True)
    ss_ref[...] += jnp.sum(m1 * m1, axis=0, keepdims=True)
    hd_ref[...] = jax.nn.sigmoid(_dotb(ea, we[...]) + be[...])


def _edge_p2(h2, src, sc2, sh2, w3, b3, wm1x, wm1e, bm1, we=None, be=None):
    ls = h2.shape[1]
    lsn = wm1x.shape[1]
    ins = [_rows(BR_E, ls), _rows(BR_E, 32), _full((1, ls)), _full((1, ls)),
           _full((ls, 19)), _full((1, 19)), _full((16, lsn)),
           _full((19, lsn)), _full((1, lsn))]
    outs = [_rows(BR_E, 19), _full((1, lsn)), _full((1, lsn))]
    oshapes = [jax.ShapeDtypeStruct((N_E, 19), bf16), _sd((1, lsn)),
               _sd((1, lsn))]
    args = [h2, src, sc2, sh2, w3, _row(b3), wm1x, wm1e, _row(bm1)]
    if we is None:
        body = _p2_body
    else:
        body = _p2_body_head
        ins += [_full((19, 1)), _full((1, 1))]
        outs.append(_rows(BR_E, 1))
        oshapes.append(_sd((N_E, 1)))
        args += [we, _row(be)]
    return pl.pallas_call(
        body, grid=(N_E // BR_E,), in_specs=ins, out_specs=outs,
        out_shape=oshapes, interpret=_INTERPRET,
    )(*args)


def _p2b_body(src, ea, wm1x, wm1e, bm1, scn, shn, *h_refs):
    m1 = (_dotb(src[...][:, :16], wm1x[...]) + _dotb(ea[...], wm1e[...])
          + bm1[...])
    h = jnp.maximum(m1 * scn[...] + shn[...], 0.0)
    w = 128 if len(h_refs) > 1 else h.shape[1]
    for i, r in enumerate(h_refs):
        r[...] = h[:, i * w:(i + 1) * w]


def _edge_p2b(src, ea, wm1x, wm1e, bm1, scn, shn):
    lsn = wm1x.shape[1]
    npart = max(1, lsn // 128)
    pw = lsn // npart
    return pl.pallas_call(
        _p2b_body,
        grid=(N_E // BR_E,),
        in_specs=[_rows(BR_E, 32), _rows(BR_E, 19), _full((16, lsn)),
                  _full((19, lsn)), _full((1, lsn)), _full((1, lsn)),
                  _full((1, lsn))],
        out_specs=[_rows(BR_E, pw)] * npart,
        out_shape=[_sd((N_E, pw))] * npart,
        interpret=_INTERPRET,
    )(src, ea, wm1x, wm1e, _row(bm1), scn, shn)


_S_CB = 15
_S_CPT = _G_NCH // _NS
_S_XTRA = _G_NCH - _S_CPT * _NS
_S_RPT = N_V // _NS


def _make_sc_scatter(ls, npart):
    nsl = ls // 16
    spp = nsl // npart

    def body(*refs):
        h_parts = refs[:npart]
        col2d, zero_hbm = refs[npart], refs[npart + 1]
        out_parts = refs[npart + 2:2 * npart + 2]
        idx_v, data_v, accum, sem = refs[2 * npart + 2:]
        cid = lax.axis_index("c")
        sid = lax.axis_index("s")

        for part in range(npart):
            h_hbm = h_parts[part]
            out_hbm = out_parts[part]

            def slice_body(k, carry):
                si = part * spp + k
                c0 = k * 16
                own = si % _NC == cid

                @pl.when(own)
                def _():
                    pltpu.sync_copy(zero_hbm.at[pl.ds(0, _S_RPT)],
                                    accum.at[pl.ds(sid * _S_RPT, _S_RPT)])

                plsc.subcore_barrier()

                def batch(b, carry2):
                    ch0 = sid + b * (_S_CB * _NS)
                    cps = [pltpu.async_copy(col2d.at[ch0 + j * _NS],
                                            idx_v.at[j], sem)
                           for j in range(_S_CB)]
                    for c in cps:
                        c.wait()
                    cps = [pltpu.async_copy(
                        h_hbm.at[pl.ds((ch0 + j * _NS) * 128, 128),
                                 pl.ds(c0, 16)],
                        data_v.at[pl.ds(j * 128, 128)], sem)
                        for j in range(_S_CB)]
                    for c in cps:
                        c.wait()
                    cps = [pltpu.async_copy(data_v.at[pl.ds(j * 128, 128)],
                                            accum.at[idx_v.at[j]], sem,
                                            add=True)
                           for j in range(_S_CB)]
                    for c in cps:
                        c.wait()
                    return carry2

                @pl.when(own)
                def _():
                    lax.fori_loop(0, _S_CPT // _S_CB, batch, 0)

                    @pl.when(sid < _S_XTRA)
                    def _():
                        ch = sid + _S_CPT * _NS
                        pltpu.sync_copy(col2d.at[ch], idx_v.at[0])
                        pltpu.async_copy(
                            h_hbm.at[pl.ds(ch * 128, 128), pl.ds(c0, 16)],
                            data_v.at[pl.ds(0, 128)], sem).wait()
                        pltpu.sync_copy(data_v.at[pl.ds(0, 128)],
                                        accum.at[idx_v.at[0]], add=True)

                plsc.subcore_barrier()

                @pl.when(own)
                def _():
                    pltpu.sync_copy(
                        accum.at[pl.ds(sid * _S_RPT, _S_RPT)],
                        out_hbm.at[pl.ds(sid * _S_RPT, _S_RPT),
                                   pl.ds(c0, 16)])

                plsc.subcore_barrier()
                return carry

            lax.fori_loop(0, spp, slice_body, 0)

    return body


def _sc_scatter(h_parts, col2d, zeros):
    npart = len(h_parts)
    pw = h_parts[0].shape[1]
    ls = npart * pw
    mesh = plsc.VectorSubcoreMesh(core_axis_name="c", subcore_axis_name="s")
    f = pl.kernel(
        _make_sc_scatter(ls, npart), mesh=mesh,
        out_type=[_sd((N_V, pw))] * npart,
        scratch_types=[pltpu.VMEM((_S_CB, 128), jnp.int32),
                       pltpu.VMEM((_S_CB * 128, 16), f32),
                       pltpu.VMEM_SHARED((N_V, 16), f32),
                       pltpu.SemaphoreType.DMA],
        compiler_params=pltpu.CompilerParams(use_tc_tiling_on_sc=False),
    )
    out = f(*h_parts, col2d, zeros)
    return out if isinstance(out, (list, tuple)) else [out]


def _sc_counts_body(col2d, ones_hbm, zero_hbm, out_hbm,
                    idx_v, ones_v, accum, sem):
    cid = lax.axis_index("c")
    sid = lax.axis_index("s")

    @pl.when(cid == 0)
    def _():
        pltpu.sync_copy(ones_hbm, ones_v)
        pltpu.sync_copy(zero_hbm.at[pl.ds(0, _S_RPT)],
                        accum.at[pl.ds(sid * _S_RPT, _S_RPT)])
        plsc.subcore_barrier()

        def batch(b, carry):
            ch0 = sid + b * (_S_CB * _NS)
            cps = [pltpu.async_copy(col2d.at[ch0 + j * _NS], idx_v.at[j],
                                    sem)
                   for j in range(_S_CB)]
            for c in cps:
                c.wait()
            cps = [pltpu.async_copy(ones_v, accum.at[idx_v.at[j]], sem,
                                    add=True)
                   for j in range(_S_CB)]
            for c in cps:
                c.wait()
            return carry

        lax.fori_loop(0, _S_CPT // _S_CB, batch, 0)

        @pl.when(sid < _S_XTRA)
        def _():
            ch = sid + _S_CPT * _NS
            pltpu.sync_copy(col2d.at[ch], idx_v.at[0])
            pltpu.sync_copy(ones_v, accum.at[idx_v.at[0]], add=True)

        plsc.subcore_barrier()
        pltpu.sync_copy(accum.at[pl.ds(sid * _S_RPT, _S_RPT)],
                        out_hbm.at[pl.ds(sid * _S_RPT, _S_RPT)])


def _sc_counts(col2d, ones, zeros):
    mesh = plsc.VectorSubcoreMesh(core_axis_name="c", subcore_axis_name="s")
    f = pl.kernel(
        _sc_counts_body, mesh=mesh,
        out_type=_sd((N_V, 16)),
        scratch_types=[pltpu.VMEM((_S_CB, 128), jnp.int32),
                       pltpu.VMEM((128, 16), f32),
                       pltpu.VMEM_SHARED((N_V, 16), f32),
                       pltpu.SemaphoreType.DMA],
        compiler_params=pltpu.CompilerParams(use_tc_tiling_on_sc=False),
    )
    return f(col2d, ones, zeros)


def _p4_body(x, cnt, wm2x, bm2, *rest):
    nseg = (len(rest) - 3) // 2
    segs = rest[:nseg]
    wm2ss = rest[nseg:2 * nseg]
    m2_ref, s_ref, ss_ref = rest[2 * nseg:]
    c = jnp.maximum(cnt[...][:, 0:1], 1.0)
    inv = 1.0 / c
    m2 = _dot(x[...], wm2x[...]) + bm2[...]
    for sg, w in zip(segs, wm2ss):
        m2 = m2 + _dot(sg[...] * inv, w[...])
    m2_ref[...] = m2

    @pl.when(pl.program_id(0) == 0)
    def _():
        s_ref[...] = jnp.zeros_like(s_ref)
        ss_ref[...] = jnp.zeros_like(ss_ref)

    s_ref[...] += jnp.sum(m2, axis=0, keepdims=True)
    ss_ref[...] += jnp.sum(m2 * m2, axis=0, keepdims=True)


def _node_p4(x, seg_parts, cnt16, wm2x, wm2s, bm2):
    pw = seg_parts[0].shape[1]
    e2 = wm2x.shape[1]
    npart = len(seg_parts)
    wparts = [wm2s[i * pw:(i + 1) * pw] for i in range(npart)]
    return pl.pallas_call(
        _p4_body,
        grid=(N_V // BR_N,),
        in_specs=([_rows(BR_N, 16), _rows(BR_N, 16), _full((16, e2)),
                   _full((1, e2))]
                  + [_rows(BR_N, pw)] * npart
                  + [_full((pw, e2))] * npart),
        out_specs=[_rows(BR_N, e2), _full((1, e2)), _full((1, e2))],
        out_shape=[_sd((N_V, e2)), _sd((1, e2)), _sd((1, e2))],
        interpret=_INTERPRET,
    )(x, cnt16, wm2x, _row(bm2), *seg_parts, *wparts)


def _p5_body_bsum(m2, oh, sc, sh, wlin, blin, x_ref, bs_ref, bc_ref):
    a = jnp.maximum(m2[...] * sc[...] + sh[...], 0.0)
    xn = jnp.maximum(_dot(a, wlin[...]) + blin[...], 0.0)
    x_ref[...] = xn

    @pl.when(pl.program_id(0) == 0)
    def _():
        bs_ref[...] = jnp.zeros_like(bs_ref)
        bc_ref[...] = jnp.zeros_like(bc_ref)

    o = oh[...]
    bs_ref[...] += lax.dot_general(o, xn, (((0,), (0,)), ((), ())),
                                   preferred_element_type=f32)
    ones = jnp.ones((o.shape[0], 1), f32)
    bc_ref[...] += lax.dot_general(o, ones, (((0,), (0,)), ((), ())),
                                   preferred_element_type=f32)


def _p5_body_head(m2, sc, sh, wlin, blin, wx, bx, x_ref, y_ref):
    a = jnp.maximum(m2[...] * sc[...] + sh[...], 0.0)
    xn = jnp.maximum(_dot(a, wlin[...]) + blin[...], 0.0)
    x_ref[...] = xn
    y_ref[...] = jax.nn.sigmoid(_dot(xn, wx[...]) + bx[...])


def _node_p5(m2, sc, sh, wlin, blin, onehot=None, wx=None, bx=None):
    e2 = m2.shape[1]
    if onehot is not None:
        return pl.pallas_call(
            _p5_body_bsum,
            grid=(N_V // BR_N,),
            in_specs=[_rows(BR_N, e2), _rows(BR_N, NG), _full((1, e2)),
                      _full((1, e2)), _full((e2, 16)), _full((1, 16))],
            out_specs=[_rows(BR_N, 16), _full((NG, 16)), _full((NG, 1))],
            out_shape=[_sd((N_V, 16)), _sd((NG, 16)), _sd((NG, 1))],
            interpret=_INTERPRET,
        )(m2, onehot, sc, sh, wlin, _row(blin))
    return pl.pallas_call(
        _p5_body_head,
        grid=(N_V // BR_N,),
        in_specs=[_rows(BR_N, e2), _full((1, e2)), _full((1, e2)),
                  _full((e2, 16)), _full((1, 16)), _full((16, 1)),
                  _full((1, 1))],
        out_specs=[_rows(BR_N, 16), _rows(BR_N, 1)],
        out_shape=[_sd((N_V, 16)), _sd((N_V, 1))],
        interpret=_INTERPRET,
    )(m2, sc, sh, wlin, _row(blin), wx, _row(bx))


def _p6_body(u, bs, bc, w1u, w1s, b1, g1, bb1, w2, b2, g2, bb2, w3, b3,
             out_ref):
    cnt = jnp.maximum(bc[...], 1.0)
    seg = bs[...] * (1.0 / cnt)
    h = _dot(u[...], w1u[...]) + _dot(seg, w1s[...]) + b1[...]

    def _bn(h, g, bb):
        mu = jnp.mean(h, axis=0, keepdims=True)
        var = jnp.mean(h * h, axis=0, keepdims=True) - mu * mu
        return (h - mu) / jnp.sqrt(var + 1e-5) * g[...] + bb[...]

    a1 = jnp.maximum(_bn(h, g1, bb1), 0.0)
    h2 = _dot(a1, w2[...]) + b2[...]
    a2 = jnp.maximum(_bn(h2, g2, bb2), 0.0)
    out_ref[...] = _dot(a2, w3[...]) + b3[...]


def _glob_p6(u, bs, bc, gp):
    w1u, w1s = gp["l1"]["w"][:7], gp["l1"]["w"][7:]
    return pl.pallas_call(
        _p6_body,
        in_specs=[_full((NG, 7)), _full((NG, 16)), _full((NG, 1)),
                  _full((7, 128)), _full((16, 128)), _full((1, 128)),
                  _full((1, 128)), _full((1, 128)), _full((128, 128)),
                  _full((1, 128)), _full((1, 128)), _full((1, 128)),
                  _full((128, 7)), _full((1, 7))],
        out_specs=_full((NG, 7)),
        out_shape=_sd((NG, 7)),
        interpret=_INTERPRET,
    )(u, bs, bc, w1u, w1s, _row(gp["l1"]["b"]), _row(gp["bn1"]["g"]),
      _row(gp["bn1"]["b"]), gp["l2"]["w"], _row(gp["l2"]["b"]),
      _row(gp["bn2"]["g"]), _row(gp["bn2"]["b"]), gp["l3"]["w"],
      _row(gp["l3"]["b"]))


_G_CPW = _G_NCH // _NW
_G_XTRA = _G_NCH - _G_CPW * _NW
_G_CB = 15
_G_NB = _G_CPW // _G_CB


def _sc_gather_body(x_hbm, row2d, col2d, src_hbm, dst_hbm,
                    idx_v, rows_v, sem):
    wid = lax.axis_index("s") * _NC + lax.axis_index("c")

    def run(ih2d, oh):
        def batch(b, carry):
            c0 = wid + b * (_G_CB * _NW)
            cps = [pltpu.async_copy(ih2d.at[c0 + j * _NW], idx_v.at[j], sem)
                   for j in range(_G_CB)]
            for c in cps:
                c.wait()
            cps = [pltpu.async_copy(x_hbm.at[idx_v.at[j]],
                                    rows_v.at[pl.ds(j * 128, 128)], sem)
                   for j in range(_G_CB)]
            for c in cps:
                c.wait()
            cps = [pltpu.async_copy(rows_v.at[pl.ds(j * 128, 128)],
                                    oh.at[pl.ds((c0 + j * _NW) * 128, 128)],
                                    sem)
                   for j in range(_G_CB)]
            for c in cps:
                c.wait()
            return carry

        lax.fori_loop(0, _G_NB, batch, 0)

        @pl.when(wid < _G_XTRA)
        def _():
            c = wid + _G_CPW * _NW
            pltpu.sync_copy(ih2d.at[c], idx_v.at[0])
            pltpu.async_copy(x_hbm.at[idx_v.at[0]],
                             rows_v.at[pl.ds(0, 128)], sem).wait()
            pltpu.sync_copy(rows_v.at[pl.ds(0, 128)],
                            oh.at[pl.ds(c * 128, 128)])

    run(row2d, src_hbm)
    run(col2d, dst_hbm)


def _sc_gather(x32, row2d, col2d):
    mesh = plsc.VectorSubcoreMesh(core_axis_name="c", subcore_axis_name="s")
    f = pl.kernel(
        _sc_gather_body, mesh=mesh,
        out_type=[jax.ShapeDtypeStruct((N_E, 32), bf16)] * 2,
        scratch_types=[pltpu.VMEM((_G_CB, 128), jnp.int32),
                       pltpu.VMEM((_G_CB * 128, 32), bf16),
                       pltpu.SemaphoreType.DMA],
        compiler_params=pltpu.CompilerParams(use_tc_tiling_on_sc=False),
    )
    return f(x32, row2d, col2d)


def kernel(x, edge_index, edge_attr, u, batch, params):
    row, col = edge_index[0], edge_index[1]
    row2d = row.reshape(N_E // 128, 128)
    col2d = col.reshape(N_E // 128, 128)
    onehot = (batch[:, None] == jnp.arange(NG, dtype=jnp.int32)[None, :]
              ).astype(f32)
    zeros = jnp.zeros((_S_RPT, 16), f32)
    ones = jnp.ones((128, 16), f32)
    cnt16 = _sc_counts(col2d, ones, zeros)
    ea = edge_attr.astype(bf16)
    y_pred = None
    head = None
    for li, name in enumerate(("ml1", "ml2", "ml3")):
        p = params[name]
        ep, np_, gp = p["edge"], p["node"], p["glob"]
        last = li == 2
        x32 = jnp.concatenate([x, x], axis=1).astype(bf16)
        src, dst = _sc_gather(x32, row2d, col2d)
        w1 = ep["l1"]["w"].astype(bf16)
        w1s, w1d, w1e = w1[:16], w1[16:32], w1[32:]
        s1, ss1 = _edge_stats1(src, dst, ea, w1s, w1d, w1e, ep["l1"]["b"])
        sc1, sh1 = _fin_stats(s1[0], ss1[0], N_E, ep["bn1"]["g"],
                              ep["bn1"]["b"])
        h2, s2, ss2 = _edge_p1(src, dst, ea, w1s, w1d, w1e, ep["l1"]["b"],
                               sc1, sh1, ep["l2"]["w"].astype(bf16),
                               ep["l2"]["b"])
        sc2, sh2 = _fin_stats(s2[0], ss2[0], N_E, ep["bn2"]["g"],
                              ep["bn2"]["b"])
        wm1 = np_["m1"]["w"].astype(bf16)
        wm1x, wm1e = wm1[:16], wm1[16:]
        if last:
            ea, sn, ssn, head = _edge_p2(
                h2, src, sc2, sh2, ep["lin"]["w"].astype(bf16),
                ep["lin"]["b"],
                wm1x, wm1e, np_["m1"]["b"],
                params["e_lin"]["w"].astype(bf16),
                params["e_lin"]["b"])
        else:
            ea, sn, ssn = _edge_p2(
                h2, src, sc2, sh2, ep["lin"]["w"].astype(bf16),
                ep["lin"]["b"],
                wm1x, wm1e, np_["m1"]["b"])
        scn, shn = _fin_stats(sn[0], ssn[0], N_E, np_["bn1"]["g"],
                              np_["bn1"]["b"])
        h_parts = _edge_p2b(src, ea, wm1x, wm1e, np_["m1"]["b"], scn, shn)
        if not isinstance(h_parts, (list, tuple)):
            h_parts = [h_parts]
        seg_parts = _sc_scatter(h_parts, col2d, zeros)
        wm2 = np_["m2"]["w"]
        m2, sm, ssm = _node_p4(x, seg_parts, cnt16, wm2[:16], wm2[16:],
                               np_["m2"]["b"])
        scm, shm = _fin_stats(sm[0], ssm[0], N_V, np_["bn2"]["g"],
                              np_["bn2"]["b"])
        if last:
            x, y_pred = _node_p5(m2, scm, shm, np_["lin"]["w"],
                                 np_["lin"]["b"], wx=params["x_lin"]["w"],
                                 bx=params["x_lin"]["b"])
        else:
            x, bs, bc = _node_p5(m2, scm, shm, np_["lin"]["w"],
                                 np_["lin"]["b"], onehot=onehot)
            u = _glob_p6(u, bs, bc, gp)
    return (y_pred, head)


# speedup vs baseline: 2.5249x; 1.0566x over previous
import functools

import jax
import jax.numpy as jnp
from jax import lax
from jax.experimental import pallas as pl
from jax.experimental.pallas import tpu as pltpu
from jax.experimental.pallas import tpu_sc as plsc

_INTERPRET = False

_NC = 2
_NS = 16
_NW = _NC * _NS
_G_NCH = 800000 // 128

N_E = 800000
N_V = 50000
NG = 64
BR_E = 8000
BR_N = 5000
f32 = jnp.float32
bf16 = jnp.bfloat16


def _row(v):
    return v.reshape(1, -1)


def _fin_stats(s, ss, n, g, bbn):
    mu = s / n
    var = ss / n - mu * mu
    isd = 1.0 / jnp.sqrt(var + 1e-5)
    sc = g * isd
    sh = bbn - mu * sc
    return _row(sc), _row(sh)


def _dot(a, b):
    return jnp.dot(a, b, preferred_element_type=f32)


def _dotb(a, b):
    return jnp.dot(a.astype(bf16), b, preferred_element_type=f32)


def _full(shape):
    return pl.BlockSpec(shape, lambda *a: tuple(0 for _ in shape))


def _rows(br, w):
    return pl.BlockSpec((br, w), lambda i: (i, 0))


def _sd(shape):
    return jax.ShapeDtypeStruct(shape, f32)


def _p0_body(src, dst, ea, w1s, w1d, w1e, b1, s_ref, ss_ref):
    h = (_dotb(src[...][:, :16], w1s[...]) + _dotb(dst[...][:, :16],
         w1d[...]) + _dotb(ea[...], w1e[...]) + b1[...])

    @pl.when(pl.program_id(0) == 0)
    def _():
        s_ref[...] = jnp.zeros_like(s_ref)
        ss_ref[...] = jnp.zeros_like(ss_ref)

    s_ref[...] += jnp.sum(h, axis=0, keepdims=True)
    ss_ref[...] += jnp.sum(h * h, axis=0, keepdims=True)


def _edge_stats1(src, dst, ea, w1s, w1d, w1e, b1):
    ls = w1s.shape[1]
    s, ss = pl.pallas_call(
        _p0_body,
        grid=(N_E // BR_E,),
        in_specs=[_rows(BR_E, 32), _rows(BR_E, 32), _rows(BR_E, 19),
                  _full((16, ls)), _full((16, ls)), _full((19, ls)),
                  _full((1, ls))],
        out_specs=[_full((1, ls)), _full((1, ls))],
        out_shape=[_sd((1, ls))] * 2,
        interpret=_INTERPRET,
    )(src, dst, ea, w1s, w1d, w1e, _row(b1))
    return s, ss


def _p1_body(src, dst, ea, w1s, w1d, w1e, b1, sc1, sh1, w2, b2,
             h2_ref, s_ref, ss_ref):
    h1 = (_dotb(src[...][:, :16], w1s[...]) + _dotb(dst[...][:, :16],
          w1d[...]) + _dotb(ea[...], w1e[...]) + b1[...])
    a1 = jnp.maximum(h1 * sc1[...] + sh1[...], 0.0)
    h2 = _dotb(a1, w2[...]) + b2[...]
    h2_ref[...] = h2.astype(bf16)

    @pl.when(pl.program_id(0) == 0)
    def _():
        s_ref[...] = jnp.zeros_like(s_ref)
        ss_ref[...] = jnp.zeros_like(ss_ref)

    s_ref[...] += jnp.sum(h2, axis=0, keepdims=True)
    ss_ref[...] += jnp.sum(h2 * h2, axis=0, keepdims=True)


def _edge_p1(src, dst, ea, w1s, w1d, w1e, b1, sc1, sh1, w2, b2):
    ls = w2.shape[1]
    return pl.pallas_call(
        _p1_body,
        grid=(N_E // BR_E,),
        in_specs=[_rows(BR_E, 32), _rows(BR_E, 32), _rows(BR_E, 19),
                  _full((16, ls)), _full((16, ls)), _full((19, ls)),
                  _full((1, ls)), _full((1, ls)), _full((1, ls)),
                  _full((ls, ls)), _full((1, ls))],
        out_specs=[_rows(BR_E, ls), _full((1, ls)), _full((1, ls))],
        out_shape=[jax.ShapeDtypeStruct((N_E, ls), bf16), _sd((1, ls)),
                   _sd((1, ls))],
        interpret=_INTERPRET,
    )(src, dst, ea, w1s, w1d, w1e, _row(b1), sc1, sh1, w2, _row(b2))


def _p2_body(h2, src, sc2, sh2, w3, b3, wm1x, wm1e, bm1,
             ea_ref, s_ref, ss_ref):
    a2 = jnp.maximum(h2[...].astype(f32) * sc2[...] + sh2[...], 0.0)
    ea = jnp.maximum(_dotb(a2, w3[...]) + b3[...], 0.0)
    ea_ref[...] = ea.astype(bf16)
    m1 = (_dotb(src[...][:, :16], wm1x[...]) + _dotb(ea, wm1e[...])
          + bm1[...])

    @pl.when(pl.program_id(0) == 0)
    def _():
        s_ref[...] = jnp.zeros_like(s_ref)
        ss_ref[...] = jnp.zeros_like(ss_ref)

    s_ref[...] += jnp.sum(m1, axis=0, keepdims=True)
    ss_ref[...] += jnp.sum(m1 * m1, axis=0, keepdims=True)


def _p2_body_head(h2, src, sc2, sh2, w3, b3, wm1x, wm1e, bm1, we, be,
                  ea_ref, s_ref, ss_ref, hd_ref):
    a2 = jnp.maximum(h2[...].astype(f32) * sc2[...] + sh2[...], 0.0)
    ea = jnp.maximum(_dotb(a2, w3[...]) + b3[...], 0.0)
    ea_ref[...] = ea.astype(bf16)
    m1 = (_dotb(src[...][:, :16], wm1x[...]) + _dotb(ea, wm1e[...])
          + bm1[...])

    @pl.when(pl.program_id(0) == 0)
    def _():
        s_ref[...] = jnp.zeros_like(s_ref)
        ss_ref[...] = jnp.zeros_like(ss_ref)

    s_ref[...] += jnp.sum(m1, axis=0, keepdims=True)
    ss_ref[...] += jnp.sum(m1 * m1, axis=0, keepdims=True)
    hd_ref[...] = jax.nn.sigmoid(_dotb(ea, we[...]) + be[...])


def _edge_p2(h2, src, sc2, sh2, w3, b3, wm1x, wm1e, bm1, we=None, be=None):
    ls = h2.shape[1]
    lsn = wm1x.shape[1]
    ins = [_rows(BR_E, ls), _rows(BR_E, 32), _full((1, ls)), _full((1, ls)),
           _full((ls, 19)), _full((1, 19)), _full((16, lsn)),
           _full((19, lsn)), _full((1, lsn))]
    outs = [_rows(BR_E, 19), _full((1, lsn)), _full((1, lsn))]
    oshapes = [jax.ShapeDtypeStruct((N_E, 19), bf16), _sd((1, lsn)),
               _sd((1, lsn))]
    args = [h2, src, sc2, sh2, w3, _row(b3), wm1x, wm1e, _row(bm1)]
    if we is None:
        body = _p2_body
    else:
        body = _p2_body_head
        ins += [_full((19, 1)), _full((1, 1))]
        outs.append(_rows(BR_E, 1))
        oshapes.append(_sd((N_E, 1)))
        args += [we, _row(be)]
    return pl.pallas_call(
        body, grid=(N_E // BR_E,), in_specs=ins, out_specs=outs,
        out_shape=oshapes, interpret=_INTERPRET,
    )(*args)


def _p2b_body(src, ea, wm1x, wm1e, bm1, scn, shn, *h_refs):
    m1 = (_dotb(src[...][:, :16], wm1x[...]) + _dotb(ea[...], wm1e[...])
          + bm1[...])
    h = jnp.maximum(m1 * scn[...] + shn[...], 0.0)
    w = 128 if len(h_refs) > 1 else h.shape[1]
    for i, r in enumerate(h_refs):
        r[...] = h[:, i * w:(i + 1) * w]


def _edge_p2b(src, ea, wm1x, wm1e, bm1, scn, shn):
    lsn = wm1x.shape[1]
    npart = max(1, lsn // 128)
    pw = lsn // npart
    return pl.pallas_call(
        _p2b_body,
        grid=(N_E // BR_E,),
        in_specs=[_rows(BR_E, 32), _rows(BR_E, 19), _full((16, lsn)),
                  _full((19, lsn)), _full((1, lsn)), _full((1, lsn)),
                  _full((1, lsn))],
        out_specs=[_rows(BR_E, pw)] * npart,
        out_shape=[_sd((N_E, pw))] * npart,
        interpret=_INTERPRET,
    )(src, ea, wm1x, wm1e, _row(bm1), scn, shn)


_S_CB = 15
_S_CPT = _G_NCH // _NS
_S_XTRA = _G_NCH - _S_CPT * _NS
_S_RPT = N_V // _NS


def _make_sc_scatter(ls, npart):
    nsl = ls // 16
    spp = nsl // npart

    def body(*refs):
        h_parts = refs[:npart]
        col2d, zero_hbm = refs[npart], refs[npart + 1]
        out_parts = refs[npart + 2:2 * npart + 2]
        idx_v, data_v, accum, sem = refs[2 * npart + 2:]
        cid = lax.axis_index("c")
        sid = lax.axis_index("s")

        for part in range(npart):
            h_hbm = h_parts[part]
            out_hbm = out_parts[part]

            def slice_body(k, carry):
                si = part * spp + k
                c0 = k * 16
                own = si % _NC == cid

                @pl.when(own)
                def _():
                    pltpu.sync_copy(zero_hbm.at[pl.ds(0, _S_RPT)],
                                    accum.at[pl.ds(sid * _S_RPT, _S_RPT)])

                plsc.subcore_barrier()

                def batch(b, carry2):
                    ch0 = sid + b * (_S_CB * _NS)
                    cps = [pltpu.async_copy(col2d.at[ch0 + j * _NS],
                                            idx_v.at[j], sem)
                           for j in range(_S_CB)]
                    for c in cps:
                        c.wait()
                    cps = [pltpu.async_copy(
                        h_hbm.at[pl.ds((ch0 + j * _NS) * 128, 128),
                                 pl.ds(c0, 16)],
                        data_v.at[pl.ds(j * 128, 128)], sem)
                        for j in range(_S_CB)]
                    for c in cps:
                        c.wait()
                    cps = [pltpu.async_copy(data_v.at[pl.ds(j * 128, 128)],
                                            accum.at[idx_v.at[j]], sem,
                                            add=True)
                           for j in range(_S_CB)]
                    for c in cps:
                        c.wait()
                    return carry2

                @pl.when(own)
                def _():
                    lax.fori_loop(0, _S_CPT // _S_CB, batch, 0)

                    @pl.when(sid < _S_XTRA)
                    def _():
                        ch = sid + _S_CPT * _NS
                        pltpu.sync_copy(col2d.at[ch], idx_v.at[0])
                        pltpu.async_copy(
                            h_hbm.at[pl.ds(ch * 128, 128), pl.ds(c0, 16)],
                            data_v.at[pl.ds(0, 128)], sem).wait()
                        pltpu.sync_copy(data_v.at[pl.ds(0, 128)],
                                        accum.at[idx_v.at[0]], add=True)

                plsc.subcore_barrier()

                @pl.when(own)
                def _():
                    pltpu.sync_copy(
                        accum.at[pl.ds(sid * _S_RPT, _S_RPT)],
                        out_hbm.at[pl.ds(sid * _S_RPT, _S_RPT),
                                   pl.ds(c0, 16)])

                plsc.subcore_barrier()
                return carry

            lax.fori_loop(0, spp, slice_body, 0)

    return body


def _sc_scatter(h_parts, col2d, zeros):
    npart = len(h_parts)
    pw = h_parts[0].shape[1]
    ls = npart * pw
    mesh = plsc.VectorSubcoreMesh(core_axis_name="c", subcore_axis_name="s")
    f = pl.kernel(
        _make_sc_scatter(ls, npart), mesh=mesh,
        out_type=[_sd((N_V, pw))] * npart,
        scratch_types=[pltpu.VMEM((_S_CB, 128), jnp.int32),
                       pltpu.VMEM((_S_CB * 128, 16), f32),
                       pltpu.VMEM_SHARED((N_V, 16), f32),
                       pltpu.SemaphoreType.DMA],
        compiler_params=pltpu.CompilerParams(use_tc_tiling_on_sc=False),
    )
    out = f(*h_parts, col2d, zeros)
    return out if isinstance(out, (list, tuple)) else [out]


def _sc_counts_body(col2d, ones_hbm, zero_hbm, out_hbm,
                    idx_v, ones_v, accum, sem):
    cid = lax.axis_index("c")
    sid = lax.axis_index("s")

    @pl.when(cid == 0)
    def _():
        pltpu.sync_copy(ones_hbm, ones_v)
        pltpu.sync_copy(zero_hbm.at[pl.ds(0, _S_RPT)],
                        accum.at[pl.ds(sid * _S_RPT, _S_RPT)])
        plsc.subcore_barrier()

        def batch(b, carry):
            ch0 = sid + b * (_S_CB * _NS)
            cps = [pltpu.async_copy(col2d.at[ch0 + j * _NS], idx_v.at[j],
                                    sem)
                   for j in range(_S_CB)]
            for c in cps:
                c.wait()
            cps = [pltpu.async_copy(ones_v, accum.at[idx_v.at[j]], sem,
                                    add=True)
                   for j in range(_S_CB)]
            for c in cps:
                c.wait()
            return carry

        lax.fori_loop(0, _S_CPT // _S_CB, batch, 0)

        @pl.when(sid < _S_XTRA)
        def _():
            ch = sid + _S_CPT * _NS
            pltpu.sync_copy(col2d.at[ch], idx_v.at[0])
            pltpu.sync_copy(ones_v, accum.at[idx_v.at[0]], add=True)

        plsc.subcore_barrier()
        pltpu.sync_copy(accum.at[pl.ds(sid * _S_RPT, _S_RPT)],
                        out_hbm.at[pl.ds(sid * _S_RPT, _S_RPT)])


def _sc_counts(col2d, ones, zeros):
    mesh = plsc.VectorSubcoreMesh(core_axis_name="c", subcore_axis_name="s")
    f = pl.kernel(
        _sc_counts_body, mesh=mesh,
        out_type=_sd((N_V, 16)),
        scratch_types=[pltpu.VMEM((_S_CB, 128), jnp.int32),
                       pltpu.VMEM((128, 16), f32),
                       pltpu.VMEM_SHARED((N_V, 16), f32),
                       pltpu.SemaphoreType.DMA],
        compiler_params=pltpu.CompilerParams(use_tc_tiling_on_sc=False),
    )
    return f(col2d, ones, zeros)


def _p4_body(x, cnt, wm2x, bm2, *rest):
    nseg = (len(rest) - 3) // 2
    segs = rest[:nseg]
    wm2ss = rest[nseg:2 * nseg]
    m2_ref, s_ref, ss_ref = rest[2 * nseg:]
    c = jnp.maximum(cnt[...][:, 0:1], 1.0)
    inv = 1.0 / c
    m2 = _dot(x[...], wm2x[...]) + bm2[...]
    for sg, w in zip(segs, wm2ss):
        m2 = m2 + _dot(sg[...] * inv, w[...])
    m2_ref[...] = m2

    @pl.when(pl.program_id(0) == 0)
    def _():
        s_ref[...] = jnp.zeros_like(s_ref)
        ss_ref[...] = jnp.zeros_like(ss_ref)

    s_ref[...] += jnp.sum(m2, axis=0, keepdims=True)
    ss_ref[...] += jnp.sum(m2 * m2, axis=0, keepdims=True)


def _node_p4(x, seg_parts, cnt16, wm2x, wm2s, bm2):
    pw = seg_parts[0].shape[1]
    e2 = wm2x.shape[1]
    npart = len(seg_parts)
    wparts = [wm2s[i * pw:(i + 1) * pw] for i in range(npart)]
    return pl.pallas_call(
        _p4_body,
        grid=(N_V // BR_N,),
        in_specs=([_rows(BR_N, 16), _rows(BR_N, 16), _full((16, e2)),
                   _full((1, e2))]
                  + [_rows(BR_N, pw)] * npart
                  + [_full((pw, e2))] * npart),
        out_specs=[_rows(BR_N, e2), _full((1, e2)), _full((1, e2))],
        out_shape=[_sd((N_V, e2)), _sd((1, e2)), _sd((1, e2))],
        interpret=_INTERPRET,
    )(x, cnt16, wm2x, _row(bm2), *seg_parts, *wparts)


def _p5_body_bsum(m2, oh, sc, sh, wlin, blin, x_ref, bs_ref, bc_ref):
    a = jnp.maximum(m2[...] * sc[...] + sh[...], 0.0)
    xn = jnp.maximum(_dot(a, wlin[...]) + blin[...], 0.0)
    x_ref[...] = xn

    @pl.when(pl.program_id(0) == 0)
    def _():
        bs_ref[...] = jnp.zeros_like(bs_ref)
        bc_ref[...] = jnp.zeros_like(bc_ref)

    o = oh[...]
    bs_ref[...] += lax.dot_general(o, xn, (((0,), (0,)), ((), ())),
                                   preferred_element_type=f32)
    ones = jnp.ones((o.shape[0], 1), f32)
    bc_ref[...] += lax.dot_general(o, ones, (((0,), (0,)), ((), ())),
                                   preferred_element_type=f32)


def _p5_body_head(m2, sc, sh, wlin, blin, wx, bx, x_ref, y_ref):
    a = jnp.maximum(m2[...] * sc[...] + sh[...], 0.0)
    xn = jnp.maximum(_dot(a, wlin[...]) + blin[...], 0.0)
    x_ref[...] = xn
    y_ref[...] = jax.nn.sigmoid(_dot(xn, wx[...]) + bx[...])


def _node_p5(m2, sc, sh, wlin, blin, onehot=None, wx=None, bx=None):
    e2 = m2.shape[1]
    if onehot is not None:
        return pl.pallas_call(
            _p5_body_bsum,
            grid=(N_V // BR_N,),
            in_specs=[_rows(BR_N, e2), _rows(BR_N, NG), _full((1, e2)),
                      _full((1, e2)), _full((e2, 16)), _full((1, 16))],
            out_specs=[_rows(BR_N, 16), _full((NG, 16)), _full((NG, 1))],
            out_shape=[_sd((N_V, 16)), _sd((NG, 16)), _sd((NG, 1))],
            interpret=_INTERPRET,
        )(m2, onehot, sc, sh, wlin, _row(blin))
    return pl.pallas_call(
        _p5_body_head,
        grid=(N_V // BR_N,),
        in_specs=[_rows(BR_N, e2), _full((1, e2)), _full((1, e2)),
                  _full((e2, 16)), _full((1, 16)), _full((16, 1)),
                  _full((1, 1))],
        out_specs=[_rows(BR_N, 16), _rows(BR_N, 1)],
        out_shape=[_sd((N_V, 16)), _sd((N_V, 1))],
        interpret=_INTERPRET,
    )(m2, sc, sh, wlin, _row(blin), wx, _row(bx))


def _p6_body(u, bs, bc, w1u, w1s, b1, g1, bb1, w2, b2, g2, bb2, w3, b3,
             out_ref):
    cnt = jnp.maximum(bc[...], 1.0)
    seg = bs[...] * (1.0 / cnt)
    h = _dot(u[...], w1u[...]) + _dot(seg, w1s[...]) + b1[...]

    def _bn(h, g, bb):
        mu = jnp.mean(h, axis=0, keepdims=True)
        var = jnp.mean(h * h, axis=0, keepdims=True) - mu * mu
        return (h - mu) / jnp.sqrt(var + 1e-5) * g[...] + bb[...]

    a1 = jnp.maximum(_bn(h, g1, bb1), 0.0)
    h2 = _dot(a1, w2[...]) + b2[...]
    a2 = jnp.maximum(_bn(h2, g2, bb2), 0.0)
    out_ref[...] = _dot(a2, w3[...]) + b3[...]


def _glob_p6(u, bs, bc, gp):
    w1u, w1s = gp["l1"]["w"][:7], gp["l1"]["w"][7:]
    return pl.pallas_call(
        _p6_body,
        in_specs=[_full((NG, 7)), _full((NG, 16)), _full((NG, 1)),
                  _full((7, 128)), _full((16, 128)), _full((1, 128)),
                  _full((1, 128)), _full((1, 128)), _full((128, 128)),
                  _full((1, 128)), _full((1, 128)), _full((1, 128)),
                  _full((128, 7)), _full((1, 7))],
        out_specs=_full((NG, 7)),
        out_shape=_sd((NG, 7)),
        interpret=_INTERPRET,
    )(u, bs, bc, w1u, w1s, _row(gp["l1"]["b"]), _row(gp["bn1"]["g"]),
      _row(gp["bn1"]["b"]), gp["l2"]["w"], _row(gp["l2"]["b"]),
      _row(gp["bn2"]["g"]), _row(gp["bn2"]["b"]), gp["l3"]["w"],
      _row(gp["l3"]["b"]))


_G_CPW = _G_NCH // _NW
_G_XTRA = _G_NCH - _G_CPW * _NW
_G_CB = 15
_G_NB = _G_CPW // _G_CB


def _sc_gather_body(x_hbm, row2d, col2d, src_hbm, dst_hbm,
                    idx_v, rows_v, sem):
    wid = lax.axis_index("s") * _NC + lax.axis_index("c")

    def run(ih2d, oh):
        def batch(b, carry):
            c0 = wid + b * (_G_CB * _NW)
            cps = [pltpu.async_copy(ih2d.at[c0 + j * _NW], idx_v.at[j], sem)
                   for j in range(_G_CB)]
            for c in cps:
                c.wait()
            cps = [pltpu.async_copy(x_hbm.at[idx_v.at[j]],
                                    rows_v.at[pl.ds(j * 128, 128)], sem)
                   for j in range(_G_CB)]
            for c in cps:
                c.wait()
            cps = [pltpu.async_copy(rows_v.at[pl.ds(j * 128, 128)],
                                    oh.at[pl.ds((c0 + j * _NW) * 128, 128)],
                                    sem)
                   for j in range(_G_CB)]
            for c in cps:
                c.wait()
            return carry

        lax.fori_loop(0, _G_NB, batch, 0)

        @pl.when(wid < _G_XTRA)
        def _():
            c = wid + _G_CPW * _NW
            pltpu.sync_copy(ih2d.at[c], idx_v.at[0])
            pltpu.async_copy(x_hbm.at[idx_v.at[0]],
                             rows_v.at[pl.ds(0, 128)], sem).wait()
            pltpu.sync_copy(rows_v.at[pl.ds(0, 128)],
                            oh.at[pl.ds(c * 128, 128)])

    run(row2d, src_hbm)
    run(col2d, dst_hbm)


def _sc_gather(x32, row2d, col2d):
    mesh = plsc.VectorSubcoreMesh(core_axis_name="c", subcore_axis_name="s")
    f = pl.kernel(
        _sc_gather_body, mesh=mesh,
        out_type=[jax.ShapeDtypeStruct((N_E, 32), bf16)] * 2,
        scratch_types=[pltpu.VMEM((_G_CB, 128), jnp.int32),
                       pltpu.VMEM((_G_CB * 128, 32), bf16),
                       pltpu.SemaphoreType.DMA],
        compiler_params=pltpu.CompilerParams(use_tc_tiling_on_sc=False),
    )
    return f(x32, row2d, col2d)


def kernel(x, edge_index, edge_attr, u, batch, params):
    row, col = edge_index[0], edge_index[1]
    row2d = row.reshape(N_E // 128, 128)
    col2d = col.reshape(N_E // 128, 128)
    onehot = (batch[:, None] == jnp.arange(NG, dtype=jnp.int32)[None, :]
              ).astype(f32)
    zeros = jnp.zeros((_S_RPT, 16), f32)
    ones = jnp.ones((128, 16), f32)
    cnt16 = _sc_counts(col2d, ones, zeros)
    ea = edge_attr.astype(bf16)
    y_pred = None
    head = None
    for li, name in enumerate(("ml1", "ml2", "ml3")):
        p = params[name]
        ep, np_, gp = p["edge"], p["node"], p["glob"]
        last = li == 2
        x32 = jnp.concatenate([x, x], axis=1).astype(bf16)
        src, dst = _sc_gather(x32, row2d, col2d)
        w1 = ep["l1"]["w"].astype(bf16)
        w1s, w1d, w1e = w1[:16], w1[16:32], w1[32:]
        s1, ss1 = _edge_stats1(src, dst, ea, w1s, w1d, w1e, ep["l1"]["b"])
        sc1, sh1 = _fin_stats(s1[0], ss1[0], N_E, ep["bn1"]["g"],
                              ep["bn1"]["b"])
        h2, s2, ss2 = _edge_p1(src, dst, ea, w1s, w1d, w1e, ep["l1"]["b"],
                               sc1, sh1, ep["l2"]["w"].astype(bf16),
                               ep["l2"]["b"])
        sc2, sh2 = _fin_stats(s2[0], ss2[0], N_E, ep["bn2"]["g"],
                              ep["bn2"]["b"])
        wm1 = np_["m1"]["w"].astype(bf16)
        wm1x, wm1e = wm1[:16], wm1[16:]
        if last:
            ea, sn, ssn, head = _edge_p2(
                h2, src, sc2, sh2, ep["lin"]["w"].astype(bf16),
                ep["lin"]["b"],
                wm1x, wm1e, np_["m1"]["b"],
                params["e_lin"]["w"].astype(bf16),
                params["e_lin"]["b"])
        else:
            ea, sn, ssn = _edge_p2(
                h2, src, sc2, sh2, ep["lin"]["w"].astype(bf16),
                ep["lin"]["b"],
                wm1x, wm1e, np_["m1"]["b"])
        scn, shn = _fin_stats(sn[0], ssn[0], N_E, np_["bn1"]["g"],
                              np_["bn1"]["b"])
        h_parts = _edge_p2b(src, ea, wm1x, wm1e, np_["m1"]["b"], scn, shn)
        if not isinstance(h_parts, (list, tuple)):
            h_parts = [h_parts]
        seg_parts = _sc_scatter(h_parts, col2d, zeros)
        wm2 = np_["m2"]["w"]
        m2, sm, ssm = _node_p4(x, seg_parts, cnt16, wm2[:16], wm2[16:],
                               np_["m2"]["b"])
        scm, shm = _fin_stats(sm[0], ssm[0], N_V, np_["bn2"]["g"],
                              np_["bn2"]["b"])
        if last:
            x, y_pred = _node_p5(m2, scm, shm, np_["lin"]["w"],
                                 np_["lin"]["b"], wx=params["x_lin"]["w"],
                                 bx=params["x_lin"]["b"])
        else:
            x, bs, bc = _node_p5(m2, scm, shm, np_["lin"]["w"],
                                 np_["lin"]["b"], onehot=onehot)
            u = _glob_p6(u, bs, bc, gp)
    return (y_pred, head)
